# Initial kernel scaffold; baseline (speedup 1.0000x reference)
import functools

import jax
import jax.numpy as jnp
from jax import lax
from jax.experimental import pallas as pl
from jax.experimental.pallas import tpu as pltpu
from jax.experimental.pallas import tpu_sc as plsc

N = 10000
E = 320000
NP = 10016
HP = 327680
TOT = 2 * HP
D = 32
DUMMY = 10008

NC, NS = 2, 16
NW = NC * NS
PER_W = TOT // NW
IDX_ROWS = TOT // 128
ROWS_W = PER_W // 128
CHUNK_ROWS = 16
CHUNK = CHUNK_ROWS * 128
N_CHUNKS = ROWS_W // CHUNK_ROWS

_sc_mesh = plsc.VectorSubcoreMesh(core_axis_name="c", subcore_axis_name="s")


@functools.partial(
    pl.kernel,
    mesh=_sc_mesh,
    out_type=jax.ShapeDtypeStruct((TOT, D), jnp.float32),
    scratch_types=[
        pltpu.VMEM((CHUNK_ROWS, 128), jnp.int32),
        pltpu.VMEM((CHUNK, D), jnp.float32),
        pltpu.SemaphoreType.DMA,
    ],
)
def _sc_gather(table_hbm, idx_hbm, out_hbm, idx_v, rows_v, sem):
    wid = lax.axis_index("s") * NC + lax.axis_index("c")

    def chunk_body(k, carry):
        row0 = wid * ROWS_W + k * CHUNK_ROWS
        pltpu.sync_copy(idx_hbm.at[pl.ds(row0, CHUNK_ROWS)], idx_v)
        cps = [
            pltpu.async_copy(
                table_hbm.at[idx_v.at[j]],
                rows_v.at[pl.ds(j * 128, 128)],
                sem,
            )
            for j in range(CHUNK_ROWS)
        ]
        for cp in cps:
            cp.wait()
        pltpu.sync_copy(
            rows_v, out_hbm.at[pl.ds(wid * PER_W + k * CHUNK, CHUNK)]
        )
        return carry

    lax.fori_loop(0, N_CHUNKS, chunk_body, 0)


@functools.partial(
    pl.kernel,
    mesh=_sc_mesh,
    out_type=jax.ShapeDtypeStruct((NC, NP, D), jnp.float32),
    scratch_types=[
        pltpu.VMEM((CHUNK_ROWS, 128), jnp.int32),
        pltpu.VMEM((CHUNK, D), jnp.float32),
        pltpu.VMEM_SHARED((NP, D), jnp.float32),
    ],
)
def _sc_scatter(msgs_hbm, idx_hbm, zeros_hbm, out_hbm, idx_v, msg_v, acc_sh):
    cid = lax.axis_index("c")
    sid = lax.axis_index("s")
    wid = sid * NC + cid

    @pl.when(sid == 0)
    def _init():
        pltpu.sync_copy(zeros_hbm, acc_sh)

    plsc.subcore_barrier()

    def chunk_body(k, carry):
        row0 = wid * ROWS_W + k * CHUNK_ROWS
        pltpu.sync_copy(idx_hbm.at[pl.ds(row0, CHUNK_ROWS)], idx_v)
        pltpu.sync_copy(
            msgs_hbm.at[pl.ds(wid * PER_W + k * CHUNK, CHUNK)], msg_v
        )
        for j in range(CHUNK_ROWS):
            pltpu.sync_copy(
                msg_v.at[pl.ds(j * 128, 128)],
                acc_sh.at[idx_v.at[j]],
                add=True,
            )
        return carry

    lax.fori_loop(0, N_CHUNKS, chunk_body, 0)
    plsc.subcore_barrier()
    rows = NP // NS
    pltpu.sync_copy(
        acc_sh.at[pl.ds(sid * rows, rows)],
        out_hbm.at[cid].at[pl.ds(sid * rows, rows)],
    )


def _setup_body(classes_ref, pos_ref, Win_ref, bin_ref, W1c_ref, h0_ref, P_ref):
    h0_ref[...] = (
        jnp.dot(classes_ref[...], Win_ref[...],
                preferred_element_type=jnp.float32)
        + bin_ref[...]
    )
    P_ref[...] = jnp.dot(pos_ref[...], W1c_ref[...],
                         preferred_element_type=jnp.float32)


def _posf_body(gp_ref, b1_ref, out_ref):
    gp = gp_ref[...]
    out_ref[...] = gp[:, D:] - gp[:, :D] + b1_ref[...]


def _mlp_body(g_ref, posf_ref, W1_ref, b1d_ref, W2_ref, b2_ref, W3_ref,
              b3_ref, out_ref):
    x = g_ref[...]
    posf = posf_ref[...]
    posterm = jnp.concatenate([posf, b1d_ref[...] - posf], axis=1)
    h1 = jnp.maximum(
        jnp.dot(x, W1_ref[...], preferred_element_type=jnp.float32)
        + posterm, 0.0)
    h2 = jnp.maximum(
        jnp.dot(h1, W2_ref[...], preferred_element_type=jnp.float32)
        + b2_ref[...], 0.0)
    out_ref[...] = (
        jnp.dot(h2, W3_ref[...], preferred_element_type=jnp.float32)
        + b3_ref[...]
    )


def _gru_body(a0_ref, a1_ref, h_ref, Wih_ref, Whh_ref, bih_ref, bhh_ref,
              out_ref):
    a = a0_ref[...] + a1_ref[...]
    h = h_ref[...]
    gi = jnp.dot(a, Wih_ref[...], preferred_element_type=jnp.float32) \
        + bih_ref[...]
    gh = jnp.dot(h, Whh_ref[...], preferred_element_type=jnp.float32) \
        + bhh_ref[...]
    r = jax.nn.sigmoid(gi[:, :D] + gh[:, :D])
    z = jax.nn.sigmoid(gi[:, D:2 * D] + gh[:, D:2 * D])
    n = jnp.tanh(gi[:, 2 * D:] + r * gh[:, 2 * D:])
    out_ref[...] = (1.0 - z) * n + z * h


_BB = 4096


def _mlp_call(G2, posf, W1cat, b1d, W2c, b2c, W3c, b3c):
    wspec = lambda shape: pl.BlockSpec(shape, lambda i: (0, 0))
    return pl.pallas_call(
        _mlp_body,
        grid=(HP // _BB,),
        in_specs=[
            pl.BlockSpec((_BB, 2 * D), lambda i: (i, 0)),
            pl.BlockSpec((_BB, D), lambda i: (i, 0)),
            wspec((2 * D, 2 * D)),
            wspec((1, D)),
            wspec((2 * D, 2 * D)),
            wspec((1, 2 * D)),
            wspec((2 * D, 2 * D)),
            wspec((1, 2 * D)),
        ],
        out_specs=pl.BlockSpec((_BB, 2 * D), lambda i: (i, 0)),
        out_shape=jax.ShapeDtypeStruct((HP, 2 * D), jnp.float32),
    )(G2, posf, W1cat, b1d, W2c, b2c, W3c, b3c)


def kernel(pos, classes, edges, W_in, b_in, W1, b1, W2, b2, W3, b3,
           nWih, nWhh, nbih, nbhh, eWih, eWhh, ebih, ebhh):
    f32 = jnp.float32
    classes_p = jnp.pad(classes, ((0, NP - N), (0, 0)))
    pos_p = jnp.pad(pos, ((0, NP - N), (0, 1)))
    W1a, W1b = W1[:D], W1[D:2 * D]
    W1c_p = jnp.pad(W1[2 * D:], ((0, 1), (0, 0)))
    W1cat = jnp.block([[W1a, W1b], [W1b, W1a]])
    zero_d = jnp.zeros((D, D), f32)
    W2c = jnp.block([[W2, zero_d], [zero_d, W2]])
    W3c = jnp.block([[W3, zero_d], [zero_d, W3]])
    b1r = b1.reshape(1, D)
    b1d = 2.0 * b1r
    b2c = jnp.concatenate([b2, b2]).reshape(1, 2 * D)
    b3c = jnp.concatenate([b3, b3]).reshape(1, 2 * D)
    binr = b_in.reshape(1, D)
    bihr = nbih.reshape(1, 3 * D)
    bhhr = nbhh.reshape(1, 3 * D)

    s_pad = jnp.pad(edges[0], (0, HP - E), constant_values=DUMMY)
    t_pad = jnp.pad(edges[1], (0, HP - E), constant_values=DUMMY)
    idx2d = jnp.stack([s_pad, t_pad], axis=1).reshape(IDX_ROWS, 128)
    zeros_acc = jnp.zeros((NP, D), f32)

    h0, P = pl.pallas_call(
        _setup_body,
        out_shape=(
            jax.ShapeDtypeStruct((NP, D), f32),
            jax.ShapeDtypeStruct((NP, D), f32),
        ),
    )(classes_p, pos_p, W_in, binr, W1c_p)

    GP = _sc_gather(P, idx2d)
    GP2 = GP.reshape(HP, 2 * D)
    posf = pl.pallas_call(
        _posf_body,
        grid=(HP // _BB,),
        in_specs=[
            pl.BlockSpec((_BB, 2 * D), lambda i: (i, 0)),
            pl.BlockSpec((1, D), lambda i: (0, 0)),
        ],
        out_specs=pl.BlockSpec((_BB, D), lambda i: (i, 0)),
        out_shape=jax.ShapeDtypeStruct((HP, D), f32),
    )(GP2, b1r)

    gru = pl.pallas_call(
        _gru_body,
        out_shape=jax.ShapeDtypeStruct((NP, D), f32),
    )

    h = h0
    for _ in range(6):
        G = _sc_gather(h, idx2d)
        msg2 = _mlp_call(G.reshape(HP, 2 * D), posf,
                         W1cat, b1d, W2c, b2c, W3c, b3c)
        acc = _sc_scatter(msg2.reshape(TOT, D), idx2d, zeros_acc)
        h = gru(acc[0], acc[1], h, nWih, nWhh, bihr, bhhr)
    return h[:N]


# speedup vs baseline: 7.9171x; 7.9171x over previous
import functools

import jax
import jax.numpy as jnp
from jax import lax
from jax.experimental import pallas as pl
from jax.experimental.pallas import tpu as pltpu
from jax.experimental.pallas import tpu_sc as plsc

N = 10000
E = 320000
NP = 10016
HP = 327680
TOT = 2 * HP
D = 32
DUMMY = 10008

NC, NS = 2, 16
NW = NC * NS
PER_W = TOT // NW
IDX_ROWS = TOT // 128
ROWS_W = PER_W // 128
CHUNK_ROWS = 16
CHUNK = CHUNK_ROWS * 128
N_CHUNKS = ROWS_W // CHUNK_ROWS

@functools.lru_cache(maxsize=1)
def _sc_kernels():
    mesh = plsc.VectorSubcoreMesh(core_axis_name="c", subcore_axis_name="s")

    @functools.partial(
        pl.kernel,
        mesh=mesh,
        out_type=jax.ShapeDtypeStruct((TOT, D), jnp.float32),
        scratch_types=[
            pltpu.VMEM((CHUNK_ROWS, 128), jnp.int32),
            pltpu.VMEM((CHUNK, D), jnp.float32),
            pltpu.SemaphoreType.DMA,
        ],
        compiler_params=pltpu.CompilerParams(use_tc_tiling_on_sc=False),
    )
    def sc_gather(table_hbm, idx_hbm, out_hbm, idx_v, rows_v, sem):
        wid = lax.axis_index("s") * NC + lax.axis_index("c")

        def chunk_body(k, carry):
            row0 = wid * ROWS_W + k * CHUNK_ROWS
            pltpu.sync_copy(idx_hbm.at[pl.ds(row0, CHUNK_ROWS)], idx_v)
            cps = [
                pltpu.async_copy(
                    table_hbm.at[idx_v.at[j]],
                    rows_v.at[pl.ds(j * 128, 128)],
                    sem,
                )
                for j in range(CHUNK_ROWS)
            ]
            for cp in cps:
                cp.wait()
            pltpu.sync_copy(
                rows_v, out_hbm.at[pl.ds(wid * PER_W + k * CHUNK, CHUNK)]
            )
            return carry

        lax.fori_loop(0, N_CHUNKS, chunk_body, 0)

    @functools.partial(
        pl.kernel,
        mesh=mesh,
        out_type=jax.ShapeDtypeStruct((NC, NP, D), jnp.float32),
        scratch_types=[
            pltpu.VMEM((CHUNK_ROWS, 128), jnp.int32),
            pltpu.VMEM((CHUNK, D), jnp.float32),
            pltpu.VMEM_SHARED((NP, D), jnp.float32),
        ],
        compiler_params=pltpu.CompilerParams(use_tc_tiling_on_sc=False),
    )
    def sc_scatter(msgs_hbm, idx_hbm, zeros_hbm, out_hbm, idx_v, msg_v,
                   acc_sh):
        cid = lax.axis_index("c")
        sid = lax.axis_index("s")
        wid = sid * NC + cid

        @pl.when(sid == 0)
        def _init():
            pltpu.sync_copy(zeros_hbm, acc_sh)

        plsc.subcore_barrier()

        def chunk_body(k, carry):
            row0 = wid * ROWS_W + k * CHUNK_ROWS
            pltpu.sync_copy(idx_hbm.at[pl.ds(row0, CHUNK_ROWS)], idx_v)
            pltpu.sync_copy(
                msgs_hbm.at[pl.ds(wid * PER_W + k * CHUNK, CHUNK)], msg_v
            )
            for j in range(CHUNK_ROWS):
                pltpu.sync_copy(
                    msg_v.at[pl.ds(j * 128, 128)],
                    acc_sh.at[idx_v.at[j]],
                    add=True,
                )
            return carry

        lax.fori_loop(0, N_CHUNKS, chunk_body, 0)
        plsc.subcore_barrier()
        rows = NP // NS
        pltpu.sync_copy(
            acc_sh.at[pl.ds(sid * rows, rows)],
            out_hbm.at[cid].at[pl.ds(sid * rows, rows)],
        )

    return sc_gather, sc_scatter


def _setup_body(classes_ref, pos_ref, Win_ref, bin_ref, W1c_ref, h0_ref, P_ref):
    h0_ref[...] = (
        jnp.dot(classes_ref[...], Win_ref[...],
                preferred_element_type=jnp.float32)
        + bin_ref[...]
    )
    P_ref[...] = jnp.dot(pos_ref[...], W1c_ref[...],
                         preferred_element_type=jnp.float32)


def _posf_body(gp_ref, b1_ref, out_ref):
    gp = gp_ref[...]
    out_ref[...] = gp[:, D:] - gp[:, :D] + b1_ref[...]


def _mlp_body(g_ref, posf_ref, W1_ref, b1d_ref, W2_ref, b2_ref, W3_ref,
              b3_ref, out_ref):
    x = g_ref[...]
    posf = posf_ref[...]
    posterm = jnp.concatenate([posf, b1d_ref[...] - posf], axis=1)
    h1 = jnp.maximum(
        jnp.dot(x, W1_ref[...], preferred_element_type=jnp.float32)
        + posterm, 0.0)
    h2 = jnp.maximum(
        jnp.dot(h1, W2_ref[...], preferred_element_type=jnp.float32)
        + b2_ref[...], 0.0)
    out_ref[...] = (
        jnp.dot(h2, W3_ref[...], preferred_element_type=jnp.float32)
        + b3_ref[...]
    )


def _gru_body(a0_ref, a1_ref, h_ref, Wih_ref, Whh_ref, bih_ref, bhh_ref,
              out_ref):
    a = a0_ref[...] + a1_ref[...]
    h = h_ref[...]
    gi = jnp.dot(a, Wih_ref[...], preferred_element_type=jnp.float32) \
        + bih_ref[...]
    gh = jnp.dot(h, Whh_ref[...], preferred_element_type=jnp.float32) \
        + bhh_ref[...]
    r = jax.nn.sigmoid(gi[:, :D] + gh[:, :D])
    z = jax.nn.sigmoid(gi[:, D:2 * D] + gh[:, D:2 * D])
    n = jnp.tanh(gi[:, 2 * D:] + r * gh[:, 2 * D:])
    out_ref[...] = (1.0 - z) * n + z * h


_BB = 4096


def _mlp_call(G2, posf, W1cat, b1d, W2c, b2c, W3c, b3c):
    wspec = lambda shape: pl.BlockSpec(shape, lambda i: (0, 0))
    return pl.pallas_call(
        _mlp_body,
        grid=(HP // _BB,),
        in_specs=[
            pl.BlockSpec((_BB, 2 * D), lambda i: (i, 0)),
            pl.BlockSpec((_BB, D), lambda i: (i, 0)),
            wspec((2 * D, 2 * D)),
            wspec((1, D)),
            wspec((2 * D, 2 * D)),
            wspec((1, 2 * D)),
            wspec((2 * D, 2 * D)),
            wspec((1, 2 * D)),
        ],
        out_specs=pl.BlockSpec((_BB, 2 * D), lambda i: (i, 0)),
        out_shape=jax.ShapeDtypeStruct((HP, 2 * D), jnp.float32),
    )(G2, posf, W1cat, b1d, W2c, b2c, W3c, b3c)


def kernel(pos, classes, edges, W_in, b_in, W1, b1, W2, b2, W3, b3,
           nWih, nWhh, nbih, nbhh, eWih, eWhh, ebih, ebhh):
    f32 = jnp.float32
    classes_p = jnp.pad(classes, ((0, NP - N), (0, 0)))
    pos_p = jnp.pad(pos, ((0, NP - N), (0, 1)))
    W1a, W1b = W1[:D], W1[D:2 * D]
    W1c_p = jnp.pad(W1[2 * D:], ((0, 1), (0, 0)))
    W1cat = jnp.block([[W1a, W1b], [W1b, W1a]])
    zero_d = jnp.zeros((D, D), f32)
    W2c = jnp.block([[W2, zero_d], [zero_d, W2]])
    W3c = jnp.block([[W3, zero_d], [zero_d, W3]])
    b1r = b1.reshape(1, D)
    b1d = 2.0 * b1r
    b2c = jnp.concatenate([b2, b2]).reshape(1, 2 * D)
    b3c = jnp.concatenate([b3, b3]).reshape(1, 2 * D)
    binr = b_in.reshape(1, D)
    bihr = nbih.reshape(1, 3 * D)
    bhhr = nbhh.reshape(1, 3 * D)

    s_pad = jnp.pad(edges[0], (0, HP - E), constant_values=DUMMY)
    t_pad = jnp.pad(edges[1], (0, HP - E), constant_values=DUMMY)
    idx2d = jnp.stack([s_pad, t_pad], axis=1).reshape(IDX_ROWS, 128)
    zeros_acc = jnp.zeros((NP, D), f32)

    h0, P = pl.pallas_call(
        _setup_body,
        out_shape=(
            jax.ShapeDtypeStruct((NP, D), f32),
            jax.ShapeDtypeStruct((NP, D), f32),
        ),
    )(classes_p, pos_p, W_in, binr, W1c_p)

    sc_gather, sc_scatter = _sc_kernels()
    GP = sc_gather(P, idx2d)
    GP2 = GP.reshape(HP, 2 * D)
    posf = pl.pallas_call(
        _posf_body,
        grid=(HP // _BB,),
        in_specs=[
            pl.BlockSpec((_BB, 2 * D), lambda i: (i, 0)),
            pl.BlockSpec((1, D), lambda i: (0, 0)),
        ],
        out_specs=pl.BlockSpec((_BB, D), lambda i: (i, 0)),
        out_shape=jax.ShapeDtypeStruct((HP, D), f32),
    )(GP2, b1r)

    gru = pl.pallas_call(
        _gru_body,
        out_shape=jax.ShapeDtypeStruct((NP, D), f32),
    )

    h = h0
    for _ in range(6):
        G = sc_gather(h, idx2d)
        msg2 = _mlp_call(G.reshape(HP, 2 * D), posf,
                         W1cat, b1d, W2c, b2c, W3c, b3c)
        acc = sc_scatter(msg2.reshape(TOT, D), idx2d, zeros_acc)
        h = gru(acc[0], acc[1], h, nWih, nWhh, bihr, bhhr)
    return h[:N]


# speedup vs baseline: 13.1728x; 1.6638x over previous
import functools

import jax
import jax.numpy as jnp
from jax import lax
from jax.experimental import pallas as pl
from jax.experimental.pallas import tpu as pltpu
from jax.experimental.pallas import tpu_sc as plsc

N = 10000
E = 320000
NP = 10016
HP = 327680
TOT = 2 * HP
D = 32
DUMMY = 10008

NC, NS = 2, 16
NW = NC * NS
PER_W = TOT // NW
IDX_ROWS = TOT // 128
ROWS_W = PER_W // 128
CHUNK_ROWS = 10
CHUNK = CHUNK_ROWS * 128
N_CHUNKS = ROWS_W // CHUNK_ROWS

@functools.lru_cache(maxsize=1)
def _sc_kernels():
    mesh = plsc.VectorSubcoreMesh(core_axis_name="c", subcore_axis_name="s")
    M_PAIRS = N_CHUNKS // 2

    @functools.partial(
        pl.kernel,
        mesh=mesh,
        out_type=jax.ShapeDtypeStruct((TOT, D), jnp.float32),
        scratch_types=[
            pltpu.VMEM((ROWS_W, 128), jnp.int32),
            pltpu.VMEM((CHUNK, D), jnp.float32),
            pltpu.VMEM((CHUNK, D), jnp.float32),
            pltpu.SemaphoreType.DMA,
            pltpu.SemaphoreType.DMA,
            pltpu.SemaphoreType.DMA,
            pltpu.SemaphoreType.DMA,
        ],
        compiler_params=pltpu.CompilerParams(use_tc_tiling_on_sc=False),
    )
    def sc_gather(table_hbm, idx_hbm, out_hbm, idx_v, rows0, rows1,
                  semg0, semg1, semo0, semo1):
        wid = lax.axis_index("s") * NC + lax.axis_index("c")
        pltpu.sync_copy(idx_hbm.at[pl.ds(wid * ROWS_W, ROWS_W)], idx_v)

        def fire(k, buf, sem):
            for j in range(CHUNK_ROWS):
                pltpu.async_copy(
                    table_hbm.at[idx_v.at[k * CHUNK_ROWS + j]],
                    buf.at[pl.ds(j * 128, 128)],
                    sem,
                )

        def drain(buf, sem):
            pltpu.make_async_copy(
                table_hbm.at[pl.ds(0, CHUNK)], buf, sem).wait()

        def store(k, buf, sem):
            pltpu.async_copy(
                buf, out_hbm.at[pl.ds(wid * PER_W + k * CHUNK, CHUNK)], sem)

        def wait_store(buf, sem):
            pltpu.make_async_copy(
                buf, out_hbm.at[pl.ds(0, CHUNK)], sem).wait()

        fire(0, rows0, semg0)

        def body(m, carry):
            @pl.when(m >= 1)
            def _():
                wait_store(rows1, semo1)
            fire(2 * m + 1, rows1, semg1)
            drain(rows0, semg0)
            store(2 * m, rows0, semo0)
            @pl.when(m < M_PAIRS - 1)
            def _():
                wait_store(rows0, semo0)
                fire(2 * m + 2, rows0, semg0)
            drain(rows1, semg1)
            store(2 * m + 1, rows1, semo1)
            return carry

        lax.fori_loop(0, M_PAIRS, body, 0)
        wait_store(rows0, semo0)
        wait_store(rows1, semo1)

    @functools.partial(
        pl.kernel,
        mesh=mesh,
        out_type=jax.ShapeDtypeStruct((NC, NP, D), jnp.float32),
        scratch_types=[
            pltpu.VMEM((ROWS_W, 128), jnp.int32),
            pltpu.VMEM((CHUNK, D), jnp.float32),
            pltpu.VMEM((CHUNK, D), jnp.float32),
            pltpu.VMEM_SHARED((NP, D), jnp.float32),
            pltpu.SemaphoreType.DMA,
            pltpu.SemaphoreType.DMA,
            pltpu.SemaphoreType.DMA,
            pltpu.SemaphoreType.DMA,
        ],
        compiler_params=pltpu.CompilerParams(use_tc_tiling_on_sc=False),
    )
    def sc_scatter(msgs_hbm, idx_hbm, zeros_hbm, out_hbm, idx_v, msg0, msg1,
                   acc_sh, seml0, seml1, sems0, sems1):
        cid = lax.axis_index("c")
        sid = lax.axis_index("s")
        wid = sid * NC + cid
        zrows = NP // NS

        def load(k, buf, sem):
            pltpu.async_copy(
                msgs_hbm.at[pl.ds(wid * PER_W + k * CHUNK, CHUNK)], buf, sem)

        def wait_load(buf, sem):
            pltpu.make_async_copy(
                msgs_hbm.at[pl.ds(0, CHUNK)], buf, sem).wait()

        def fire_sc(k, buf, sem):
            for j in range(CHUNK_ROWS):
                pltpu.async_copy(
                    buf.at[pl.ds(j * 128, 128)],
                    acc_sh.at[idx_v.at[k * CHUNK_ROWS + j]],
                    sem,
                    add=True,
                )

        def drain_sc(buf, sem):
            pltpu.make_async_copy(
                msgs_hbm.at[pl.ds(0, CHUNK)], buf, sem).wait()

        load(0, msg0, seml0)
        pltpu.sync_copy(idx_hbm.at[pl.ds(wid * ROWS_W, ROWS_W)], idx_v)
        pltpu.sync_copy(
            zeros_hbm.at[pl.ds(sid * zrows, zrows)],
            acc_sh.at[pl.ds(sid * zrows, zrows)],
        )
        plsc.subcore_barrier()

        def body(m, carry):
            wait_load(msg0, seml0)
            fire_sc(2 * m, msg0, sems0)
            @pl.when(m >= 1)
            def _():
                drain_sc(msg1, sems1)
            load(2 * m + 1, msg1, seml1)
            wait_load(msg1, seml1)
            fire_sc(2 * m + 1, msg1, sems1)
            drain_sc(msg0, sems0)
            @pl.when(m < M_PAIRS - 1)
            def _():
                load(2 * m + 2, msg0, seml0)
            return carry

        lax.fori_loop(0, M_PAIRS, body, 0)
        drain_sc(msg1, sems1)
        plsc.subcore_barrier()
        pltpu.sync_copy(
            acc_sh.at[pl.ds(sid * zrows, zrows)],
            out_hbm.at[cid].at[pl.ds(sid * zrows, zrows)],
        )

    return sc_gather, sc_scatter


def _setup_body(classes_ref, pos_ref, Win_ref, bin_ref, W1c_ref, h0_ref, P_ref):
    h0_ref[...] = (
        jnp.dot(classes_ref[...], Win_ref[...],
                preferred_element_type=jnp.float32)
        + bin_ref[...]
    )
    P_ref[...] = jnp.dot(pos_ref[...], W1c_ref[...],
                         preferred_element_type=jnp.float32)


def _posf_body(gp_ref, b1_ref, out_ref):
    gp = gp_ref[...]
    b1 = b1_ref[...]
    dA = gp[:, D:2 * D] - gp[:, :D]
    dB = gp[:, 3 * D:] - gp[:, 2 * D:3 * D]
    out_ref[...] = jnp.concatenate(
        [dA + b1, b1 - dA, dB + b1, b1 - dB], axis=1)


def _mlp_body(g_ref, posf_ref, W1_ref, W2_ref, b2_ref, W3_ref,
              b3_ref, out_ref):
    x = g_ref[...]
    h1 = jnp.maximum(
        jnp.dot(x, W1_ref[...], preferred_element_type=jnp.float32)
        + posf_ref[...], 0.0)
    h2 = jnp.maximum(
        jnp.dot(h1, W2_ref[...], preferred_element_type=jnp.float32)
        + b2_ref[...], 0.0)
    out_ref[...] = (
        jnp.dot(h2, W3_ref[...], preferred_element_type=jnp.float32)
        + b3_ref[...]
    )


def _gru_body(a0_ref, a1_ref, h_ref, Wih_ref, Whh_ref, bih_ref, bhh_ref,
              out_ref):
    a = a0_ref[...] + a1_ref[...]
    h = h_ref[...]
    gi = jnp.dot(a, Wih_ref[...], preferred_element_type=jnp.float32) \
        + bih_ref[...]
    gh = jnp.dot(h, Whh_ref[...], preferred_element_type=jnp.float32) \
        + bhh_ref[...]
    r = jax.nn.sigmoid(gi[:, :D] + gh[:, :D])
    z = jax.nn.sigmoid(gi[:, D:2 * D] + gh[:, D:2 * D])
    n = jnp.tanh(gi[:, 2 * D:] + r * gh[:, 2 * D:])
    out_ref[...] = (1.0 - z) * n + z * h


_BB = 4096
TOT4 = TOT // 4


def _mlp_call(G4, posterm, W1c4, W2c4, b2c4, W3c4, b3c4):
    wspec = lambda shape: pl.BlockSpec(shape, lambda i: (0, 0))
    return pl.pallas_call(
        _mlp_body,
        grid=(TOT4 // _BB,),
        in_specs=[
            pl.BlockSpec((_BB, 4 * D), lambda i: (i, 0)),
            pl.BlockSpec((_BB, 4 * D), lambda i: (i, 0)),
            wspec((4 * D, 4 * D)),
            wspec((4 * D, 4 * D)),
            wspec((1, 4 * D)),
            wspec((4 * D, 4 * D)),
            wspec((1, 4 * D)),
        ],
        out_specs=pl.BlockSpec((_BB, 4 * D), lambda i: (i, 0)),
        out_shape=jax.ShapeDtypeStruct((TOT4, 4 * D), jnp.float32),
    )(G4, posterm, W1c4, W2c4, b2c4, W3c4, b3c4)


def kernel(pos, classes, edges, W_in, b_in, W1, b1, W2, b2, W3, b3,
           nWih, nWhh, nbih, nbhh, eWih, eWhh, ebih, ebhh):
    f32 = jnp.float32
    classes_p = jnp.pad(classes, ((0, NP - N), (0, 0)))
    pos_p = jnp.pad(pos, ((0, NP - N), (0, 1)))
    W1a, W1b = W1[:D], W1[D:2 * D]
    W1c_p = jnp.pad(W1[2 * D:], ((0, 1), (0, 0)))
    W1cat = jnp.block([[W1a, W1b], [W1b, W1a]])
    eye2 = jnp.eye(2, dtype=f32)
    eye4 = jnp.eye(4, dtype=f32)
    W1c4 = jnp.kron(eye2, W1cat)
    W2c4 = jnp.kron(eye4, W2)
    W3c4 = jnp.kron(eye4, W3)
    b1r = b1.reshape(1, D)
    b2c4 = jnp.tile(b2, 4).reshape(1, 4 * D)
    b3c4 = jnp.tile(b3, 4).reshape(1, 4 * D)
    binr = b_in.reshape(1, D)
    bihr = nbih.reshape(1, 3 * D)
    bhhr = nbhh.reshape(1, 3 * D)

    s_pad = jnp.pad(edges[0], (0, HP - E), constant_values=DUMMY)
    t_pad = jnp.pad(edges[1], (0, HP - E), constant_values=DUMMY)
    idx2d = jnp.stack([s_pad, t_pad], axis=1).reshape(IDX_ROWS, 128)
    zeros_acc = jnp.zeros((NP, D), f32)

    h0, P = pl.pallas_call(
        _setup_body,
        out_shape=(
            jax.ShapeDtypeStruct((NP, D), f32),
            jax.ShapeDtypeStruct((NP, D), f32),
        ),
    )(classes_p, pos_p, W_in, binr, W1c_p)

    sc_gather, sc_scatter = _sc_kernels()
    GP = sc_gather(P, idx2d)
    GP4 = GP.reshape(TOT4, 4 * D)
    posterm = pl.pallas_call(
        _posf_body,
        grid=(TOT4 // _BB,),
        in_specs=[
            pl.BlockSpec((_BB, 4 * D), lambda i: (i, 0)),
            pl.BlockSpec((1, D), lambda i: (0, 0)),
        ],
        out_specs=pl.BlockSpec((_BB, 4 * D), lambda i: (i, 0)),
        out_shape=jax.ShapeDtypeStruct((TOT4, 4 * D), f32),
    )(GP4, b1r)

    gru = pl.pallas_call(
        _gru_body,
        out_shape=jax.ShapeDtypeStruct((NP, D), f32),
    )

    h = h0
    for _ in range(6):
        G = sc_gather(h, idx2d)
        msg4 = _mlp_call(G.reshape(TOT4, 4 * D), posterm,
                         W1c4, W2c4, b2c4, W3c4, b3c4)
        acc = sc_scatter(msg4.reshape(TOT, D), idx2d, zeros_acc)
        h = gru(acc[0], acc[1], h, nWih, nWhh, bihr, bhhr)
    return h[:N]


# speedup vs baseline: 14.9519x; 1.1351x over previous
import functools

import jax
import jax.numpy as jnp
from jax import lax
from jax.experimental import pallas as pl
from jax.experimental.pallas import tpu as pltpu
from jax.experimental.pallas import tpu_sc as plsc

N = 10000
E = 320000
NP = 10016
HP = 327680
TOT = 2 * HP
D = 32
DUMMY = 10008

NC, NS = 2, 16
NW = NC * NS
PER_W = TOT // NW
IDX_ROWS = TOT // 128
ROWS_W = PER_W // 128
CHUNK_ROWS = 10
CHUNK = CHUNK_ROWS * 128
N_CHUNKS = ROWS_W // CHUNK_ROWS

@functools.lru_cache(maxsize=1)
def _sc_kernels():
    mesh = plsc.VectorSubcoreMesh(core_axis_name="c", subcore_axis_name="s")
    M_PAIRS = N_CHUNKS // 2

    @functools.partial(
        pl.kernel,
        mesh=mesh,
        out_type=jax.ShapeDtypeStruct((TOT, D), jnp.float32),
        scratch_types=[
            pltpu.VMEM((ROWS_W, 128), jnp.int32),
            pltpu.VMEM((CHUNK, D), jnp.float32),
            pltpu.VMEM((CHUNK, D), jnp.float32),
            pltpu.SemaphoreType.DMA,
            pltpu.SemaphoreType.DMA,
            pltpu.SemaphoreType.DMA,
            pltpu.SemaphoreType.DMA,
        ],
        compiler_params=pltpu.CompilerParams(use_tc_tiling_on_sc=False),
    )
    def sc_gather(table_hbm, idx_hbm, out_hbm, idx_v, rows0, rows1,
                  semg0, semg1, semo0, semo1):
        wid = lax.axis_index("s") * NC + lax.axis_index("c")
        pltpu.sync_copy(idx_hbm.at[pl.ds(wid * ROWS_W, ROWS_W)], idx_v)

        def fire(k, buf, sem):
            for j in range(CHUNK_ROWS):
                pltpu.async_copy(
                    table_hbm.at[idx_v.at[k * CHUNK_ROWS + j]],
                    buf.at[pl.ds(j * 128, 128)],
                    sem,
                )

        def drain(buf, sem):
            pltpu.make_async_copy(
                table_hbm.at[pl.ds(0, CHUNK)], buf, sem).wait()

        def store(k, buf, sem):
            pltpu.async_copy(
                buf, out_hbm.at[pl.ds(wid * PER_W + k * CHUNK, CHUNK)], sem)

        def wait_store(buf, sem):
            pltpu.make_async_copy(
                buf, out_hbm.at[pl.ds(0, CHUNK)], sem).wait()

        fire(0, rows0, semg0)

        def body(m, carry):
            @pl.when(m >= 1)
            def _():
                wait_store(rows1, semo1)
            fire(2 * m + 1, rows1, semg1)
            drain(rows0, semg0)
            store(2 * m, rows0, semo0)
            @pl.when(m < M_PAIRS - 1)
            def _():
                wait_store(rows0, semo0)
                fire(2 * m + 2, rows0, semg0)
            drain(rows1, semg1)
            store(2 * m + 1, rows1, semo1)
            return carry

        lax.fori_loop(0, M_PAIRS, body, 0)
        wait_store(rows0, semo0)
        wait_store(rows1, semo1)

    @functools.partial(
        pl.kernel,
        mesh=mesh,
        out_type=jax.ShapeDtypeStruct((NC, NP, D), jnp.float32),
        scratch_types=[
            pltpu.VMEM((ROWS_W, 128), jnp.int32),
            pltpu.VMEM((CHUNK, D), jnp.float32),
            pltpu.VMEM((CHUNK, D), jnp.float32),
            pltpu.VMEM_SHARED((NP, D), jnp.float32),
            pltpu.SemaphoreType.DMA,
            pltpu.SemaphoreType.DMA,
            pltpu.SemaphoreType.DMA,
            pltpu.SemaphoreType.DMA,
        ],
        compiler_params=pltpu.CompilerParams(use_tc_tiling_on_sc=False),
    )
    def sc_scatter(msgs_hbm, idx_hbm, zeros_hbm, out_hbm, idx_v, msg0, msg1,
                   acc_sh, seml0, seml1, sems0, sems1):
        cid = lax.axis_index("c")
        sid = lax.axis_index("s")
        wid = sid * NC + cid
        zrows = NP // NS

        def load(k, buf, sem):
            pltpu.async_copy(
                msgs_hbm.at[pl.ds(wid * PER_W + k * CHUNK, CHUNK)], buf, sem)

        def wait_load(buf, sem):
            pltpu.make_async_copy(
                msgs_hbm.at[pl.ds(0, CHUNK)], buf, sem).wait()

        def fire_sc(k, buf, sem):
            for j in range(CHUNK_ROWS):
                pltpu.async_copy(
                    buf.at[pl.ds(j * 128, 128)],
                    acc_sh.at[idx_v.at[k * CHUNK_ROWS + j]],
                    sem,
                    add=True,
                )

        def drain_sc(buf, sem):
            pltpu.make_async_copy(
                msgs_hbm.at[pl.ds(0, CHUNK)], buf, sem).wait()

        load(0, msg0, seml0)
        pltpu.sync_copy(idx_hbm.at[pl.ds(wid * ROWS_W, ROWS_W)], idx_v)
        pltpu.sync_copy(
            zeros_hbm.at[pl.ds(sid * zrows, zrows)],
            acc_sh.at[pl.ds(sid * zrows, zrows)],
        )
        plsc.subcore_barrier()

        def body(m, carry):
            wait_load(msg0, seml0)
            fire_sc(2 * m, msg0, sems0)
            @pl.when(m >= 1)
            def _():
                drain_sc(msg1, sems1)
            load(2 * m + 1, msg1, seml1)
            wait_load(msg1, seml1)
            fire_sc(2 * m + 1, msg1, sems1)
            drain_sc(msg0, sems0)
            @pl.when(m < M_PAIRS - 1)
            def _():
                load(2 * m + 2, msg0, seml0)
            return carry

        lax.fori_loop(0, M_PAIRS, body, 0)
        drain_sc(msg1, sems1)
        plsc.subcore_barrier()
        pltpu.sync_copy(
            acc_sh.at[pl.ds(sid * zrows, zrows)],
            out_hbm.at[cid].at[pl.ds(sid * zrows, zrows)],
        )

    return sc_gather, sc_scatter


def _setup_body(classes_ref, pos_ref, Win_ref, bin_ref, W1c_ref, h0_ref, P_ref):
    h0_ref[...] = (
        jnp.dot(classes_ref[...], Win_ref[...],
                preferred_element_type=jnp.float32)
        + bin_ref[...]
    )
    P_ref[...] = jnp.dot(pos_ref[...], W1c_ref[...],
                         preferred_element_type=jnp.float32)


def _posf_body(gps_ref, gpt_ref, b1t_ref, out_ref):
    out_ref[...] = gpt_ref[...] - gps_ref[...] + b1t_ref[...]


def _mlp_body(gs_ref, gt_ref, posf_ref, W1A_ref, W1B_ref, W2_ref, b2_ref,
              W3_ref, b3_ref, b1d_ref, out_ref):
    hf = pl.program_id(0).astype(jnp.float32)
    posf = posf_ref[...]
    term = posf + hf * (b1d_ref[...] - 2.0 * posf)
    h1 = jnp.maximum(
        jnp.dot(gs_ref[...], W1A_ref[...], preferred_element_type=jnp.float32)
        + jnp.dot(gt_ref[...], W1B_ref[...],
                  preferred_element_type=jnp.float32)
        + term, 0.0)
    h2 = jnp.maximum(
        jnp.dot(h1, W2_ref[...], preferred_element_type=jnp.float32)
        + b2_ref[...], 0.0)
    out_ref[...] = (
        jnp.dot(h2, W3_ref[...], preferred_element_type=jnp.float32)
        + b3_ref[...]
    )


def _gru_body(a0_ref, a1_ref, h_ref, Wih_ref, Whh_ref, bih_ref, bhh_ref,
              out_ref):
    a = a0_ref[...] + a1_ref[...]
    h = h_ref[...]
    gi = jnp.dot(a, Wih_ref[...], preferred_element_type=jnp.float32) \
        + bih_ref[...]
    gh = jnp.dot(h, Whh_ref[...], preferred_element_type=jnp.float32) \
        + bhh_ref[...]
    r = jax.nn.sigmoid(gi[:, :D] + gh[:, :D])
    z = jax.nn.sigmoid(gi[:, D:2 * D] + gh[:, D:2 * D])
    n = jnp.tanh(gi[:, 2 * D:] + r * gh[:, 2 * D:])
    out_ref[...] = (1.0 - z) * n + z * h


_BB = 4096
TOT4 = TOT // 4
TOT8 = TOT // 8
_NB8 = TOT8 // _BB


def _mlp_call(G4, posterm, W1A4, W1B4, W2c4, b2c4, W3c4, b3c4, b1d4):
    wspec = lambda shape: pl.BlockSpec(shape, lambda h, i: (0, 0))
    return pl.pallas_call(
        _mlp_body,
        grid=(2, _NB8),
        in_specs=[
            pl.BlockSpec((_BB, 4 * D), lambda h, i: (h * _NB8 + i, 0)),
            pl.BlockSpec((_BB, 4 * D), lambda h, i: ((1 - h) * _NB8 + i, 0)),
            pl.BlockSpec((_BB, 4 * D), lambda h, i: (i, 0)),
            wspec((4 * D, 4 * D)),
            wspec((4 * D, 4 * D)),
            wspec((4 * D, 4 * D)),
            wspec((1, 4 * D)),
            wspec((4 * D, 4 * D)),
            wspec((1, 4 * D)),
            wspec((1, 4 * D)),
        ],
        out_specs=pl.BlockSpec((_BB, 4 * D), lambda h, i: (h * _NB8 + i, 0)),
        out_shape=jax.ShapeDtypeStruct((TOT4, 4 * D), jnp.float32),
    )(G4, G4, posterm, W1A4, W1B4, W2c4, b2c4, W3c4, b3c4, b1d4)


def kernel(pos, classes, edges, W_in, b_in, W1, b1, W2, b2, W3, b3,
           nWih, nWhh, nbih, nbhh, eWih, eWhh, ebih, ebhh):
    f32 = jnp.float32
    classes_p = jnp.pad(classes, ((0, NP - N), (0, 0)))
    pos_p = jnp.pad(pos, ((0, NP - N), (0, 1)))
    W1a, W1b = W1[:D], W1[D:2 * D]
    W1c_p = jnp.pad(W1[2 * D:], ((0, 1), (0, 0)))
    eye4 = jnp.eye(4, dtype=f32)
    W1A4 = jnp.kron(eye4, W1a)
    W1B4 = jnp.kron(eye4, W1b)
    W2c4 = jnp.kron(eye4, W2)
    W3c4 = jnp.kron(eye4, W3)
    b1r = b1.reshape(1, D)
    b1t4 = jnp.tile(b1, 4).reshape(1, 4 * D)
    b1d4 = 2.0 * b1t4
    b2c4 = jnp.tile(b2, 4).reshape(1, 4 * D)
    b3c4 = jnp.tile(b3, 4).reshape(1, 4 * D)
    binr = b_in.reshape(1, D)
    bihr = nbih.reshape(1, 3 * D)
    bhhr = nbhh.reshape(1, 3 * D)

    s_pad = jnp.pad(edges[0], (0, HP - E), constant_values=DUMMY)
    t_pad = jnp.pad(edges[1], (0, HP - E), constant_values=DUMMY)
    idx2d = jnp.concatenate([s_pad, t_pad]).reshape(IDX_ROWS, 128)
    zeros_acc = jnp.zeros((NP, D), f32)

    h0, P = pl.pallas_call(
        _setup_body,
        out_shape=(
            jax.ShapeDtypeStruct((NP, D), f32),
            jax.ShapeDtypeStruct((NP, D), f32),
        ),
    )(classes_p, pos_p, W_in, binr, W1c_p)

    sc_gather, sc_scatter = _sc_kernels()
    GP = sc_gather(P, idx2d)
    GP4 = GP.reshape(TOT4, 4 * D)
    posterm = pl.pallas_call(
        _posf_body,
        grid=(_NB8,),
        in_specs=[
            pl.BlockSpec((_BB, 4 * D), lambda i: (i, 0)),
            pl.BlockSpec((_BB, 4 * D), lambda i: (_NB8 + i, 0)),
            pl.BlockSpec((1, 4 * D), lambda i: (0, 0)),
        ],
        out_specs=pl.BlockSpec((_BB, 4 * D), lambda i: (i, 0)),
        out_shape=jax.ShapeDtypeStruct((TOT8, 4 * D), f32),
    )(GP4, GP4, b1t4)

    gru = pl.pallas_call(
        _gru_body,
        out_shape=jax.ShapeDtypeStruct((NP, D), f32),
    )

    h = h0
    for _ in range(6):
        G = sc_gather(h, idx2d)
        msg4 = _mlp_call(G.reshape(TOT4, 4 * D), posterm,
                         W1A4, W1B4, W2c4, b2c4, W3c4, b3c4, b1d4)
        acc = sc_scatter(msg4.reshape(TOT, D), idx2d, zeros_acc)
        h = gru(acc[0], acc[1], h, nWih, nWhh, bihr, bhhr)
    return h[:N]


# speedup vs baseline: 15.1084x; 1.0105x over previous
import functools

import jax
import jax.numpy as jnp
from jax import lax
from jax.experimental import pallas as pl
from jax.experimental.pallas import tpu as pltpu
from jax.experimental.pallas import tpu_sc as plsc

N = 10000
E = 320000
NP = 10016
HP = 327680
TOT = 2 * HP
D = 32
DUMMY = 10008

NC, NS = 2, 16
NW = NC * NS
PER_W = TOT // NW
IDX_ROWS = TOT // 128
ROWS_W = PER_W // 128
CHUNK_ROWS = 10
CHUNK = CHUNK_ROWS * 128
N_CHUNKS = ROWS_W // CHUNK_ROWS

@functools.lru_cache(maxsize=2)
def _sc_kernels(total):
    mesh = plsc.VectorSubcoreMesh(core_axis_name="c", subcore_axis_name="s")
    PER_W = total // NW
    ROWS_W = PER_W // 128
    N_CHUNKS = ROWS_W // CHUNK_ROWS
    M_PAIRS = N_CHUNKS // 2

    @functools.partial(
        pl.kernel,
        mesh=mesh,
        out_type=jax.ShapeDtypeStruct((total, D), jnp.float32),
        scratch_types=[
            pltpu.VMEM((ROWS_W, 128), jnp.int32),
            pltpu.VMEM((CHUNK, D), jnp.float32),
            pltpu.VMEM((CHUNK, D), jnp.float32),
            pltpu.SemaphoreType.DMA,
            pltpu.SemaphoreType.DMA,
            pltpu.SemaphoreType.DMA,
            pltpu.SemaphoreType.DMA,
        ],
        compiler_params=pltpu.CompilerParams(use_tc_tiling_on_sc=False),
    )
    def sc_gather(table_hbm, idx_hbm, out_hbm, idx_v, rows0, rows1,
                  semg0, semg1, semo0, semo1):
        wid = lax.axis_index("s") * NC + lax.axis_index("c")
        pltpu.sync_copy(idx_hbm.at[pl.ds(wid * ROWS_W, ROWS_W)], idx_v)

        def fire(k, buf, sem):
            for j in range(CHUNK_ROWS):
                pltpu.async_copy(
                    table_hbm.at[idx_v.at[k * CHUNK_ROWS + j]],
                    buf.at[pl.ds(j * 128, 128)],
                    sem,
                )

        def drain(buf, sem):
            pltpu.make_async_copy(
                table_hbm.at[pl.ds(0, CHUNK)], buf, sem).wait()

        def store(k, buf, sem):
            pltpu.async_copy(
                buf, out_hbm.at[pl.ds(wid * PER_W + k * CHUNK, CHUNK)], sem)

        def wait_store(buf, sem):
            pltpu.make_async_copy(
                buf, out_hbm.at[pl.ds(0, CHUNK)], sem).wait()

        fire(0, rows0, semg0)

        def body(m, carry):
            @pl.when(m >= 1)
            def _():
                wait_store(rows1, semo1)
            fire(2 * m + 1, rows1, semg1)
            drain(rows0, semg0)
            store(2 * m, rows0, semo0)
            @pl.when(m < M_PAIRS - 1)
            def _():
                wait_store(rows0, semo0)
                fire(2 * m + 2, rows0, semg0)
            drain(rows1, semg1)
            store(2 * m + 1, rows1, semo1)
            return carry

        lax.fori_loop(0, M_PAIRS, body, 0)
        wait_store(rows0, semo0)
        wait_store(rows1, semo1)

    @functools.partial(
        pl.kernel,
        mesh=mesh,
        out_type=jax.ShapeDtypeStruct((NC, NP, D), jnp.float32),
        scratch_types=[
            pltpu.VMEM((ROWS_W, 128), jnp.int32),
            pltpu.VMEM((CHUNK, D), jnp.float32),
            pltpu.VMEM((CHUNK, D), jnp.float32),
            pltpu.VMEM_SHARED((NP, D), jnp.float32),
            pltpu.SemaphoreType.DMA,
            pltpu.SemaphoreType.DMA,
            pltpu.SemaphoreType.DMA,
            pltpu.SemaphoreType.DMA,
        ],
        compiler_params=pltpu.CompilerParams(use_tc_tiling_on_sc=False),
    )
    def sc_scatter(msgs_hbm, idx_hbm, zeros_hbm, out_hbm, idx_v, msg0, msg1,
                   acc_sh, seml0, seml1, sems0, sems1):
        cid = lax.axis_index("c")
        sid = lax.axis_index("s")
        wid = sid * NC + cid
        zrows = NP // NS

        def load(k, buf, sem):
            pltpu.async_copy(
                msgs_hbm.at[pl.ds(wid * PER_W + k * CHUNK, CHUNK)], buf, sem)

        def wait_load(buf, sem):
            pltpu.make_async_copy(
                msgs_hbm.at[pl.ds(0, CHUNK)], buf, sem).wait()

        def fire_sc(k, buf, sem):
            for j in range(CHUNK_ROWS):
                pltpu.async_copy(
                    buf.at[pl.ds(j * 128, 128)],
                    acc_sh.at[idx_v.at[k * CHUNK_ROWS + j]],
                    sem,
                    add=True,
                )

        def drain_sc(buf, sem):
            pltpu.make_async_copy(
                msgs_hbm.at[pl.ds(0, CHUNK)], buf, sem).wait()

        load(0, msg0, seml0)
        pltpu.sync_copy(idx_hbm.at[pl.ds(wid * ROWS_W, ROWS_W)], idx_v)
        pltpu.sync_copy(
            zeros_hbm.at[pl.ds(sid * zrows, zrows)],
            acc_sh.at[pl.ds(sid * zrows, zrows)],
        )
        plsc.subcore_barrier()

        def body(m, carry):
            wait_load(msg0, seml0)
            fire_sc(2 * m, msg0, sems0)
            @pl.when(m >= 1)
            def _():
                drain_sc(msg1, sems1)
            load(2 * m + 1, msg1, seml1)
            wait_load(msg1, seml1)
            fire_sc(2 * m + 1, msg1, sems1)
            drain_sc(msg0, sems0)
            @pl.when(m < M_PAIRS - 1)
            def _():
                load(2 * m + 2, msg0, seml0)
            return carry

        lax.fori_loop(0, M_PAIRS, body, 0)
        drain_sc(msg1, sems1)
        plsc.subcore_barrier()
        pltpu.sync_copy(
            acc_sh.at[pl.ds(sid * zrows, zrows)],
            out_hbm.at[cid].at[pl.ds(sid * zrows, zrows)],
        )

    return sc_gather, sc_scatter


def _setup_body(classes_ref, pos_ref, Win_ref, bin_ref, W1c_ref, h0_ref, P_ref):
    h0_ref[...] = (
        jnp.dot(classes_ref[...], Win_ref[...],
                preferred_element_type=jnp.float32)
        + bin_ref[...]
    )
    P_ref[...] = jnp.dot(pos_ref[...], W1c_ref[...],
                         preferred_element_type=jnp.float32)


def _posf_body(gps_ref, gpt_ref, b1t_ref, out_ref):
    out_ref[...] = gpt_ref[...] - gps_ref[...] + b1t_ref[...]


def _mlp_body(gs_ref, gt_ref, posf_ref, W1A_ref, W1B_ref, W2_ref, b2_ref,
              W3_ref, b3_ref, b1d_ref, out_ref):
    hf = pl.program_id(0).astype(jnp.float32)
    posf = posf_ref[...]
    term = posf + hf * (b1d_ref[...] - 2.0 * posf)
    h1 = jnp.maximum(
        jnp.dot(gs_ref[...], W1A_ref[...], preferred_element_type=jnp.float32)
        + jnp.dot(gt_ref[...], W1B_ref[...],
                  preferred_element_type=jnp.float32)
        + term, 0.0)
    h2 = jnp.maximum(
        jnp.dot(h1, W2_ref[...], preferred_element_type=jnp.float32)
        + b2_ref[...], 0.0)
    out_ref[...] = (
        jnp.dot(h2, W3_ref[...], preferred_element_type=jnp.float32)
        + b3_ref[...]
    )


def _gru_body(a0_ref, a1_ref, a2_ref, a3_ref, h_ref, Wih_ref, Whh_ref,
              bih_ref, bhh_ref, out_ref):
    a = (a0_ref[...] + a1_ref[...]) + (a2_ref[...] + a3_ref[...])
    h = h_ref[...]
    gi = jnp.dot(a, Wih_ref[...], preferred_element_type=jnp.float32) \
        + bih_ref[...]
    gh = jnp.dot(h, Whh_ref[...], preferred_element_type=jnp.float32) \
        + bhh_ref[...]
    r = jax.nn.sigmoid(gi[:, :D] + gh[:, :D])
    z = jax.nn.sigmoid(gi[:, D:2 * D] + gh[:, D:2 * D])
    n = jnp.tanh(gi[:, 2 * D:] + r * gh[:, 2 * D:])
    out_ref[...] = (1.0 - z) * n + z * h


_BB = 4096
TOT4 = TOT // 4
TOT8 = TOT // 8
_NB8 = TOT8 // _BB
_NBG = _NB8 // 2


def _mlp_call(G4, posterm, W1A4, W1B4, W2c4, b2c4, W3c4, b3c4, b1d4, goff):
    wspec = lambda shape: pl.BlockSpec(shape, lambda h, i: (0, 0))
    return pl.pallas_call(
        _mlp_body,
        grid=(2, _NBG),
        in_specs=[
            pl.BlockSpec((_BB, 4 * D),
                         lambda h, i: (h * _NB8 + goff + i, 0)),
            pl.BlockSpec((_BB, 4 * D),
                         lambda h, i: ((1 - h) * _NB8 + goff + i, 0)),
            pl.BlockSpec((_BB, 4 * D), lambda h, i: (goff + i, 0)),
            wspec((4 * D, 4 * D)),
            wspec((4 * D, 4 * D)),
            wspec((4 * D, 4 * D)),
            wspec((1, 4 * D)),
            wspec((4 * D, 4 * D)),
            wspec((1, 4 * D)),
            wspec((1, 4 * D)),
        ],
        out_specs=pl.BlockSpec((_BB, 4 * D), lambda h, i: (h * _NBG + i, 0)),
        out_shape=jax.ShapeDtypeStruct((TOT8, 4 * D), jnp.float32),
    )(G4, G4, posterm, W1A4, W1B4, W2c4, b2c4, W3c4, b3c4, b1d4)


def kernel(pos, classes, edges, W_in, b_in, W1, b1, W2, b2, W3, b3,
           nWih, nWhh, nbih, nbhh, eWih, eWhh, ebih, ebhh):
    f32 = jnp.float32
    classes_p = jnp.pad(classes, ((0, NP - N), (0, 0)))
    pos_p = jnp.pad(pos, ((0, NP - N), (0, 1)))
    W1a, W1b = W1[:D], W1[D:2 * D]
    W1c_p = jnp.pad(W1[2 * D:], ((0, 1), (0, 0)))
    eye4 = jnp.eye(4, dtype=f32)
    W1A4 = jnp.kron(eye4, W1a)
    W1B4 = jnp.kron(eye4, W1b)
    W2c4 = jnp.kron(eye4, W2)
    W3c4 = jnp.kron(eye4, W3)
    b1r = b1.reshape(1, D)
    b1t4 = jnp.tile(b1, 4).reshape(1, 4 * D)
    b1d4 = 2.0 * b1t4
    b2c4 = jnp.tile(b2, 4).reshape(1, 4 * D)
    b3c4 = jnp.tile(b3, 4).reshape(1, 4 * D)
    binr = b_in.reshape(1, D)
    bihr = nbih.reshape(1, 3 * D)
    bhhr = nbhh.reshape(1, 3 * D)

    s_pad = jnp.pad(edges[0], (0, HP - E), constant_values=DUMMY)
    t_pad = jnp.pad(edges[1], (0, HP - E), constant_values=DUMMY)
    idx2d = jnp.concatenate([s_pad, t_pad]).reshape(IDX_ROWS, 128)
    HP2 = HP // 2
    idxA = jnp.concatenate([s_pad[:HP2], t_pad[:HP2]]).reshape(
        IDX_ROWS // 2, 128)
    idxB = jnp.concatenate([s_pad[HP2:], t_pad[HP2:]]).reshape(
        IDX_ROWS // 2, 128)
    zeros_acc = jnp.zeros((NP, D), f32)

    h0, P = pl.pallas_call(
        _setup_body,
        out_shape=(
            jax.ShapeDtypeStruct((NP, D), f32),
            jax.ShapeDtypeStruct((NP, D), f32),
        ),
    )(classes_p, pos_p, W_in, binr, W1c_p)

    sc_gather, _ = _sc_kernels(TOT)
    _, sc_scatter_h = _sc_kernels(TOT // 2)
    GP = sc_gather(P, idx2d)
    GP4 = GP.reshape(TOT4, 4 * D)
    posterm = pl.pallas_call(
        _posf_body,
        grid=(_NB8,),
        in_specs=[
            pl.BlockSpec((_BB, 4 * D), lambda i: (i, 0)),
            pl.BlockSpec((_BB, 4 * D), lambda i: (_NB8 + i, 0)),
            pl.BlockSpec((1, 4 * D), lambda i: (0, 0)),
        ],
        out_specs=pl.BlockSpec((_BB, 4 * D), lambda i: (i, 0)),
        out_shape=jax.ShapeDtypeStruct((TOT8, 4 * D), f32),
    )(GP4, GP4, b1t4)

    gru = pl.pallas_call(
        _gru_body,
        out_shape=jax.ShapeDtypeStruct((NP, D), f32),
    )

    h = h0
    for _ in range(6):
        G = sc_gather(h, idx2d)
        G4 = G.reshape(TOT4, 4 * D)
        mA = _mlp_call(G4, posterm, W1A4, W1B4, W2c4, b2c4, W3c4, b3c4,
                       b1d4, 0)
        accA = sc_scatter_h(mA.reshape(TOT // 2, D), idxA, zeros_acc)
        mB = _mlp_call(G4, posterm, W1A4, W1B4, W2c4, b2c4, W3c4, b3c4,
                       b1d4, _NBG)
        accB = sc_scatter_h(mB.reshape(TOT // 2, D), idxB, zeros_acc)
        h = gru(accA[0], accA[1], accB[0], accB[1], h,
                nWih, nWhh, bihr, bhhr)
    return h[:N]


# speedup vs baseline: 25.9339x; 1.7165x over previous
import functools

import jax
import jax.numpy as jnp
from jax import lax
from jax.experimental import pallas as pl
from jax.experimental.pallas import tpu as pltpu
from jax.experimental.pallas import tpu_sc as plsc

N = 10000
E = 320000
NP = 10016
HP = 327680
TOT = 2 * HP
D = 32
DUMMY = 10008

NC, NS = 2, 16
NW = NC * NS
PER_W = TOT // NW
IDX_ROWS = TOT // 128
ROWS_W = PER_W // 128
CHUNK_ROWS = 10
CHUNK = CHUNK_ROWS * 128
N_CHUNKS = ROWS_W // CHUNK_ROWS

@functools.lru_cache(maxsize=2)
def _sc_kernels(total):
    mesh = plsc.VectorSubcoreMesh(core_axis_name="c", subcore_axis_name="s")
    PER_W = total // NW
    ROWS_W = PER_W // 128
    N_CHUNKS = ROWS_W // CHUNK_ROWS
    M_PAIRS = N_CHUNKS // 2

    @functools.partial(
        pl.kernel,
        mesh=mesh,
        out_type=jax.ShapeDtypeStruct((total, D), jnp.float32),
        scratch_types=[
            pltpu.VMEM((ROWS_W, 128), jnp.int32),
            pltpu.VMEM((CHUNK, D), jnp.float32),
            pltpu.VMEM((CHUNK, D), jnp.float32),
            pltpu.VMEM_SHARED((NP, D), jnp.float32),
            pltpu.SemaphoreType.DMA,
            pltpu.SemaphoreType.DMA,
            pltpu.SemaphoreType.DMA,
            pltpu.SemaphoreType.DMA,
        ],
        compiler_params=pltpu.CompilerParams(use_tc_tiling_on_sc=False),
    )
    def sc_gather(table_hbm, idx_hbm, out_hbm, idx_v, rows0, rows1,
                  table_sh, semg0, semg1, semo0, semo1):
        sid = lax.axis_index("s")
        wid = sid * NC + lax.axis_index("c")
        zrows = NP // NS
        pltpu.sync_copy(
            table_hbm.at[pl.ds(sid * zrows, zrows)],
            table_sh.at[pl.ds(sid * zrows, zrows)],
        )
        pltpu.sync_copy(idx_hbm.at[pl.ds(wid * ROWS_W, ROWS_W)], idx_v)
        plsc.subcore_barrier()

        def fire(k, buf, sem):
            for j in range(CHUNK_ROWS):
                pltpu.async_copy(
                    table_sh.at[idx_v.at[k * CHUNK_ROWS + j]],
                    buf.at[pl.ds(j * 128, 128)],
                    sem,
                )

        def drain(buf, sem):
            pltpu.make_async_copy(
                table_hbm.at[pl.ds(0, CHUNK)], buf, sem).wait()

        def store(k, buf, sem):
            pltpu.async_copy(
                buf, out_hbm.at[pl.ds(wid * PER_W + k * CHUNK, CHUNK)], sem)

        def wait_store(buf, sem):
            pltpu.make_async_copy(
                buf, out_hbm.at[pl.ds(0, CHUNK)], sem).wait()

        fire(0, rows0, semg0)

        def body(m, carry):
            @pl.when(m >= 1)
            def _():
                wait_store(rows1, semo1)
            fire(2 * m + 1, rows1, semg1)
            drain(rows0, semg0)
            store(2 * m, rows0, semo0)
            @pl.when(m < M_PAIRS - 1)
            def _():
                wait_store(rows0, semo0)
                fire(2 * m + 2, rows0, semg0)
            drain(rows1, semg1)
            store(2 * m + 1, rows1, semo1)
            return carry

        lax.fori_loop(0, M_PAIRS, body, 0)
        wait_store(rows0, semo0)
        wait_store(rows1, semo1)

    @functools.partial(
        pl.kernel,
        mesh=mesh,
        out_type=jax.ShapeDtypeStruct((NC, NP, D), jnp.float32),
        scratch_types=[
            pltpu.VMEM((ROWS_W, 128), jnp.int32),
            pltpu.VMEM((CHUNK, D), jnp.float32),
            pltpu.VMEM((CHUNK, D), jnp.float32),
            pltpu.VMEM_SHARED((NP, D), jnp.float32),
            pltpu.SemaphoreType.DMA,
            pltpu.SemaphoreType.DMA,
            pltpu.SemaphoreType.DMA,
            pltpu.SemaphoreType.DMA,
        ],
        compiler_params=pltpu.CompilerParams(use_tc_tiling_on_sc=False),
    )
    def sc_scatter(msgs_hbm, idx_hbm, zeros_hbm, out_hbm, idx_v, msg0, msg1,
                   acc_sh, seml0, seml1, sems0, sems1):
        cid = lax.axis_index("c")
        sid = lax.axis_index("s")
        wid = sid * NC + cid
        zrows = NP // NS

        def load(k, buf, sem):
            pltpu.async_copy(
                msgs_hbm.at[pl.ds(wid * PER_W + k * CHUNK, CHUNK)], buf, sem)

        def wait_load(buf, sem):
            pltpu.make_async_copy(
                msgs_hbm.at[pl.ds(0, CHUNK)], buf, sem).wait()

        def fire_sc(k, buf, sem):
            for j in range(CHUNK_ROWS):
                pltpu.async_copy(
                    buf.at[pl.ds(j * 128, 128)],
                    acc_sh.at[idx_v.at[k * CHUNK_ROWS + j]],
                    sem,
                    add=True,
                )

        def drain_sc(buf, sem):
            pltpu.make_async_copy(
                msgs_hbm.at[pl.ds(0, CHUNK)], buf, sem).wait()

        load(0, msg0, seml0)
        pltpu.sync_copy(idx_hbm.at[pl.ds(wid * ROWS_W, ROWS_W)], idx_v)
        pltpu.sync_copy(
            zeros_hbm.at[pl.ds(sid * zrows, zrows)],
            acc_sh.at[pl.ds(sid * zrows, zrows)],
        )
        plsc.subcore_barrier()

        def body(m, carry):
            wait_load(msg0, seml0)
            fire_sc(2 * m, msg0, sems0)
            @pl.when(m >= 1)
            def _():
                drain_sc(msg1, sems1)
            load(2 * m + 1, msg1, seml1)
            wait_load(msg1, seml1)
            fire_sc(2 * m + 1, msg1, sems1)
            drain_sc(msg0, sems0)
            @pl.when(m < M_PAIRS - 1)
            def _():
                load(2 * m + 2, msg0, seml0)
            return carry

        lax.fori_loop(0, M_PAIRS, body, 0)
        drain_sc(msg1, sems1)
        plsc.subcore_barrier()
        pltpu.sync_copy(
            acc_sh.at[pl.ds(sid * zrows, zrows)],
            out_hbm.at[cid].at[pl.ds(sid * zrows, zrows)],
        )

    return sc_gather, sc_scatter


def _setup_body(classes_ref, pos_ref, Win_ref, bin_ref, W1c_ref, h0_ref, P_ref):
    h0_ref[...] = (
        jnp.dot(classes_ref[...], Win_ref[...],
                preferred_element_type=jnp.float32)
        + bin_ref[...]
    )
    P_ref[...] = jnp.dot(pos_ref[...], W1c_ref[...],
                         preferred_element_type=jnp.float32)


def _posf_body(gps_ref, gpt_ref, b1t_ref, out_ref):
    out_ref[...] = gpt_ref[...] - gps_ref[...] + b1t_ref[...]


def _mlp_body(gs_ref, gt_ref, posf_ref, W1A_ref, W1B_ref, W2_ref, b2_ref,
              W3_ref, b3_ref, b1d_ref, out_ref):
    hf = pl.program_id(0).astype(jnp.float32)
    posf = posf_ref[...]
    term = posf + hf * (b1d_ref[...] - 2.0 * posf)
    h1 = jnp.maximum(
        jnp.dot(gs_ref[...], W1A_ref[...], preferred_element_type=jnp.float32)
        + jnp.dot(gt_ref[...], W1B_ref[...],
                  preferred_element_type=jnp.float32)
        + term, 0.0)
    h2 = jnp.maximum(
        jnp.dot(h1, W2_ref[...], preferred_element_type=jnp.float32)
        + b2_ref[...], 0.0)
    out_ref[...] = (
        jnp.dot(h2, W3_ref[...], preferred_element_type=jnp.float32)
        + b3_ref[...]
    )


def _gru_body(a0_ref, a1_ref, a2_ref, a3_ref, h_ref, Wih_ref, Whh_ref,
              bih_ref, bhh_ref, out_ref):
    a = (a0_ref[...] + a1_ref[...]) + (a2_ref[...] + a3_ref[...])
    h = h_ref[...]
    gi = jnp.dot(a, Wih_ref[...], preferred_element_type=jnp.float32) \
        + bih_ref[...]
    gh = jnp.dot(h, Whh_ref[...], preferred_element_type=jnp.float32) \
        + bhh_ref[...]
    r = jax.nn.sigmoid(gi[:, :D] + gh[:, :D])
    z = jax.nn.sigmoid(gi[:, D:2 * D] + gh[:, D:2 * D])
    n = jnp.tanh(gi[:, 2 * D:] + r * gh[:, 2 * D:])
    out_ref[...] = (1.0 - z) * n + z * h


_BB = 4096
TOT4 = TOT // 4
TOT8 = TOT // 8
_NB8 = TOT8 // _BB
_NBG = _NB8 // 2


def _mlp_call(G4, posterm, W1A4, W1B4, W2c4, b2c4, W3c4, b3c4, b1d4, goff):
    wspec = lambda shape: pl.BlockSpec(shape, lambda h, i: (0, 0))
    return pl.pallas_call(
        _mlp_body,
        grid=(2, _NBG),
        in_specs=[
            pl.BlockSpec((_BB, 4 * D),
                         lambda h, i: (h * _NB8 + goff + i, 0)),
            pl.BlockSpec((_BB, 4 * D),
                         lambda h, i: ((1 - h) * _NB8 + goff + i, 0)),
            pl.BlockSpec((_BB, 4 * D), lambda h, i: (goff + i, 0)),
            wspec((4 * D, 4 * D)),
            wspec((4 * D, 4 * D)),
            wspec((4 * D, 4 * D)),
            wspec((1, 4 * D)),
            wspec((4 * D, 4 * D)),
            wspec((1, 4 * D)),
            wspec((1, 4 * D)),
        ],
        out_specs=pl.BlockSpec((_BB, 4 * D), lambda h, i: (h * _NBG + i, 0)),
        out_shape=jax.ShapeDtypeStruct((TOT8, 4 * D), jnp.float32),
    )(G4, G4, posterm, W1A4, W1B4, W2c4, b2c4, W3c4, b3c4, b1d4)


def kernel(pos, classes, edges, W_in, b_in, W1, b1, W2, b2, W3, b3,
           nWih, nWhh, nbih, nbhh, eWih, eWhh, ebih, ebhh):
    f32 = jnp.float32
    classes_p = jnp.pad(classes, ((0, NP - N), (0, 0)))
    pos_p = jnp.pad(pos, ((0, NP - N), (0, 1)))
    W1a, W1b = W1[:D], W1[D:2 * D]
    W1c_p = jnp.pad(W1[2 * D:], ((0, 1), (0, 0)))
    eye4 = jnp.eye(4, dtype=f32)
    W1A4 = jnp.kron(eye4, W1a)
    W1B4 = jnp.kron(eye4, W1b)
    W2c4 = jnp.kron(eye4, W2)
    W3c4 = jnp.kron(eye4, W3)
    b1r = b1.reshape(1, D)
    b1t4 = jnp.tile(b1, 4).reshape(1, 4 * D)
    b1d4 = 2.0 * b1t4
    b2c4 = jnp.tile(b2, 4).reshape(1, 4 * D)
    b3c4 = jnp.tile(b3, 4).reshape(1, 4 * D)
    binr = b_in.reshape(1, D)
    bihr = nbih.reshape(1, 3 * D)
    bhhr = nbhh.reshape(1, 3 * D)

    s_pad = jnp.pad(edges[0], (0, HP - E), constant_values=DUMMY)
    t_pad = jnp.pad(edges[1], (0, HP - E), constant_values=DUMMY)
    idx2d = jnp.concatenate([s_pad, t_pad]).reshape(IDX_ROWS, 128)
    HP2 = HP // 2
    idxA = jnp.concatenate([s_pad[:HP2], t_pad[:HP2]]).reshape(
        IDX_ROWS // 2, 128)
    idxB = jnp.concatenate([s_pad[HP2:], t_pad[HP2:]]).reshape(
        IDX_ROWS // 2, 128)
    zeros_acc = jnp.zeros((NP, D), f32)

    h0, P = pl.pallas_call(
        _setup_body,
        out_shape=(
            jax.ShapeDtypeStruct((NP, D), f32),
            jax.ShapeDtypeStruct((NP, D), f32),
        ),
    )(classes_p, pos_p, W_in, binr, W1c_p)

    sc_gather, _ = _sc_kernels(TOT)
    _, sc_scatter_h = _sc_kernels(TOT // 2)
    GP = sc_gather(P, idx2d)
    GP4 = GP.reshape(TOT4, 4 * D)
    posterm = pl.pallas_call(
        _posf_body,
        grid=(_NB8,),
        in_specs=[
            pl.BlockSpec((_BB, 4 * D), lambda i: (i, 0)),
            pl.BlockSpec((_BB, 4 * D), lambda i: (_NB8 + i, 0)),
            pl.BlockSpec((1, 4 * D), lambda i: (0, 0)),
        ],
        out_specs=pl.BlockSpec((_BB, 4 * D), lambda i: (i, 0)),
        out_shape=jax.ShapeDtypeStruct((TOT8, 4 * D), f32),
    )(GP4, GP4, b1t4)

    gru = pl.pallas_call(
        _gru_body,
        out_shape=jax.ShapeDtypeStruct((NP, D), f32),
    )

    h = h0
    for _ in range(6):
        G = sc_gather(h, idx2d)
        G4 = G.reshape(TOT4, 4 * D)
        mA = _mlp_call(G4, posterm, W1A4, W1B4, W2c4, b2c4, W3c4, b3c4,
                       b1d4, 0)
        accA = sc_scatter_h(mA.reshape(TOT // 2, D), idxA, zeros_acc)
        mB = _mlp_call(G4, posterm, W1A4, W1B4, W2c4, b2c4, W3c4, b3c4,
                       b1d4, _NBG)
        accB = sc_scatter_h(mB.reshape(TOT // 2, D), idxB, zeros_acc)
        h = gru(accA[0], accA[1], accB[0], accB[1], h,
                nWih, nWhh, bihr, bhhr)
    return h[:N]


# speedup vs baseline: 29.3553x; 1.1319x over previous
import functools

import jax
import jax.numpy as jnp
from jax import lax
from jax.experimental import pallas as pl
from jax.experimental.pallas import tpu as pltpu
from jax.experimental.pallas import tpu_sc as plsc

N = 10000
E = 320000
NP = 10016
HP = 327680
TOT = 2 * HP
D = 32
DUMMY = 10008

NC, NS = 2, 16
NW = NC * NS
PER_W = TOT // NW
IDX_ROWS = TOT // 128
ROWS_W = PER_W // 128
CHUNK_ROWS = 10
CHUNK = CHUNK_ROWS * 128
N_CHUNKS = ROWS_W // CHUNK_ROWS

@functools.lru_cache(maxsize=2)
def _sc_kernels(total):
    mesh = plsc.VectorSubcoreMesh(core_axis_name="c", subcore_axis_name="s")
    PER_W = total // NW
    ROWS_W = PER_W // 128
    N_CHUNKS = ROWS_W // CHUNK_ROWS
    M_PAIRS = N_CHUNKS // 2

    @functools.partial(
        pl.kernel,
        mesh=mesh,
        out_type=jax.ShapeDtypeStruct((total, D), jnp.float32),
        scratch_types=[
            pltpu.VMEM((ROWS_W, 128), jnp.int32),
            pltpu.VMEM((CHUNK, D), jnp.float32),
            pltpu.VMEM((CHUNK, D), jnp.float32),
            pltpu.VMEM_SHARED((NP, D), jnp.float32),
            pltpu.SemaphoreType.DMA,
            pltpu.SemaphoreType.DMA,
            pltpu.SemaphoreType.DMA,
            pltpu.SemaphoreType.DMA,
        ],
        compiler_params=pltpu.CompilerParams(use_tc_tiling_on_sc=False),
    )
    def sc_gather(table_hbm, idx_hbm, out_hbm, idx_v, rows0, rows1,
                  table_sh, semg0, semg1, semo0, semo1):
        sid = lax.axis_index("s")
        wid = sid * NC + lax.axis_index("c")
        zrows = NP // NS
        pltpu.sync_copy(
            table_hbm.at[pl.ds(sid * zrows, zrows)],
            table_sh.at[pl.ds(sid * zrows, zrows)],
        )
        pltpu.sync_copy(idx_hbm.at[pl.ds(wid * ROWS_W, ROWS_W)], idx_v)
        plsc.subcore_barrier()

        def fire(k, buf, sem):
            for j in range(CHUNK_ROWS):
                pltpu.async_copy(
                    table_sh.at[idx_v.at[k * CHUNK_ROWS + j]],
                    buf.at[pl.ds(j * 128, 128)],
                    sem,
                )

        def drain(buf, sem):
            pltpu.make_async_copy(
                table_hbm.at[pl.ds(0, CHUNK)], buf, sem).wait()

        def store(k, buf, sem):
            pltpu.async_copy(
                buf, out_hbm.at[pl.ds(wid * PER_W + k * CHUNK, CHUNK)], sem)

        def wait_store(buf, sem):
            pltpu.make_async_copy(
                buf, out_hbm.at[pl.ds(0, CHUNK)], sem).wait()

        fire(0, rows0, semg0)

        def body(m, carry):
            @pl.when(m >= 1)
            def _():
                wait_store(rows1, semo1)
            fire(2 * m + 1, rows1, semg1)
            drain(rows0, semg0)
            store(2 * m, rows0, semo0)
            @pl.when(m < M_PAIRS - 1)
            def _():
                wait_store(rows0, semo0)
                fire(2 * m + 2, rows0, semg0)
            drain(rows1, semg1)
            store(2 * m + 1, rows1, semo1)
            return carry

        lax.fori_loop(0, M_PAIRS, body, 0)
        wait_store(rows0, semo0)
        wait_store(rows1, semo1)

    @functools.partial(
        pl.kernel,
        mesh=mesh,
        out_type=jax.ShapeDtypeStruct((NC, NP, D), jnp.float32),
        scratch_types=[
            pltpu.VMEM((ROWS_W, 128), jnp.int32),
            pltpu.VMEM((CHUNK, D), jnp.float32),
            pltpu.VMEM((CHUNK, D), jnp.float32),
            pltpu.VMEM_SHARED((NP, D), jnp.float32),
            pltpu.SemaphoreType.DMA,
            pltpu.SemaphoreType.DMA,
            pltpu.SemaphoreType.DMA,
            pltpu.SemaphoreType.DMA,
        ],
        compiler_params=pltpu.CompilerParams(use_tc_tiling_on_sc=False),
    )
    def sc_scatter(msgs_hbm, idx_hbm, zeros_hbm, out_hbm, idx_v, msg0, msg1,
                   acc_sh, seml0, seml1, sems0, sems1):
        cid = lax.axis_index("c")
        sid = lax.axis_index("s")
        wid = sid * NC + cid
        zrows = NP // NS

        def load(k, buf, sem):
            pltpu.async_copy(
                msgs_hbm.at[pl.ds(wid * PER_W + k * CHUNK, CHUNK)], buf, sem)

        def wait_load(buf, sem):
            pltpu.make_async_copy(
                msgs_hbm.at[pl.ds(0, CHUNK)], buf, sem).wait()

        def fire_sc(k, buf, sem):
            for j in range(CHUNK_ROWS):
                pltpu.async_copy(
                    buf.at[pl.ds(j * 128, 128)],
                    acc_sh.at[idx_v.at[k * CHUNK_ROWS + j]],
                    sem,
                    add=True,
                )

        def drain_sc(buf, sem):
            pltpu.make_async_copy(
                msgs_hbm.at[pl.ds(0, CHUNK)], buf, sem).wait()

        load(0, msg0, seml0)
        pltpu.sync_copy(idx_hbm.at[pl.ds(wid * ROWS_W, ROWS_W)], idx_v)
        pltpu.sync_copy(
            zeros_hbm.at[pl.ds(sid * zrows, zrows)],
            acc_sh.at[pl.ds(sid * zrows, zrows)],
        )
        plsc.subcore_barrier()

        def body(m, carry):
            wait_load(msg0, seml0)
            fire_sc(2 * m, msg0, sems0)
            @pl.when(m >= 1)
            def _():
                drain_sc(msg1, sems1)
            load(2 * m + 1, msg1, seml1)
            wait_load(msg1, seml1)
            fire_sc(2 * m + 1, msg1, sems1)
            drain_sc(msg0, sems0)
            @pl.when(m < M_PAIRS - 1)
            def _():
                load(2 * m + 2, msg0, seml0)
            return carry

        lax.fori_loop(0, M_PAIRS, body, 0)
        drain_sc(msg1, sems1)
        plsc.subcore_barrier()
        pltpu.sync_copy(
            acc_sh.at[pl.ds(sid * zrows, zrows)],
            out_hbm.at[cid].at[pl.ds(sid * zrows, zrows)],
        )

    return sc_gather, sc_scatter


def _setup_body(classes_ref, pos_ref, Win_ref, bin_ref, W1c_ref, h0_ref, P_ref):
    h0_ref[...] = (
        jnp.dot(classes_ref[...], Win_ref[...],
                preferred_element_type=jnp.float32)
        + bin_ref[...]
    )
    P_ref[...] = jnp.dot(pos_ref[...], W1c_ref[...],
                         preferred_element_type=jnp.float32)


def _posf_body(gps_ref, gpt_ref, b1t_ref, out_ref):
    out_ref[...] = gpt_ref[...] - gps_ref[...] + b1t_ref[...]


def _mlp_body(gs_ref, gt_ref, posf_ref, W1A_ref, W1B_ref, W2_ref, b2_ref,
              W3_ref, b3_ref, b1d_ref, out_ref):
    hf = pl.program_id(0).astype(jnp.float32)
    posf = posf_ref[...]
    term = posf + hf * (b1d_ref[...] - 2.0 * posf)
    h1 = jnp.maximum(
        jnp.dot(gs_ref[...], W1A_ref[...], preferred_element_type=jnp.float32)
        + jnp.dot(gt_ref[...], W1B_ref[...],
                  preferred_element_type=jnp.float32)
        + term, 0.0)
    h2 = jnp.maximum(
        jnp.dot(h1, W2_ref[...], preferred_element_type=jnp.float32)
        + b2_ref[...], 0.0)
    out_ref[...] = (
        jnp.dot(h2, W3_ref[...], preferred_element_type=jnp.float32)
        + b3_ref[...]
    )


def _gru_body(aA0_ref, aA1_ref, aB0_ref, aB1_ref, h_ref, Wih_ref, Whh_ref,
              bih_ref, bhh_ref, out_ref):
    a = (aA0_ref[0] + aA1_ref[0]) + (aB0_ref[0] + aB1_ref[0])
    h = h_ref[...]
    gi = jnp.dot(a, Wih_ref[...], preferred_element_type=jnp.float32) \
        + bih_ref[...]
    gh = jnp.dot(h, Whh_ref[...], preferred_element_type=jnp.float32) \
        + bhh_ref[...]
    outs = []
    for j in range(4):
        b = 3 * D * j
        hj = h[:, D * j:D * (j + 1)]
        r = jax.nn.sigmoid(gi[:, b:b + D] + gh[:, b:b + D])
        z = jax.nn.sigmoid(gi[:, b + D:b + 2 * D] + gh[:, b + D:b + 2 * D])
        n = jnp.tanh(gi[:, b + 2 * D:b + 3 * D]
                     + r * gh[:, b + 2 * D:b + 3 * D])
        outs.append((1.0 - z) * n + z * hj)
    out_ref[...] = jnp.concatenate(outs, axis=1)


_BB = 4096
TOT4 = TOT // 4
TOT8 = TOT // 8
_NB8 = TOT8 // _BB
_NBG = _NB8 // 2


def _mlp_call(G4, posterm, W1A4, W1B4, W2c4, b2c4, W3c4, b3c4, b1d4, goff):
    wspec = lambda shape: pl.BlockSpec(shape, lambda h, i: (0, 0))
    return pl.pallas_call(
        _mlp_body,
        grid=(2, _NBG),
        in_specs=[
            pl.BlockSpec((_BB, 4 * D),
                         lambda h, i: (h * _NB8 + goff + i, 0)),
            pl.BlockSpec((_BB, 4 * D),
                         lambda h, i: ((1 - h) * _NB8 + goff + i, 0)),
            pl.BlockSpec((_BB, 4 * D), lambda h, i: (goff + i, 0)),
            wspec((4 * D, 4 * D)),
            wspec((4 * D, 4 * D)),
            wspec((4 * D, 4 * D)),
            wspec((1, 4 * D)),
            wspec((4 * D, 4 * D)),
            wspec((1, 4 * D)),
            wspec((1, 4 * D)),
        ],
        out_specs=pl.BlockSpec((_BB, 4 * D), lambda h, i: (h * _NBG + i, 0)),
        out_shape=jax.ShapeDtypeStruct((TOT8, 4 * D), jnp.float32),
    )(G4, G4, posterm, W1A4, W1B4, W2c4, b2c4, W3c4, b3c4, b1d4)


def kernel(pos, classes, edges, W_in, b_in, W1, b1, W2, b2, W3, b3,
           nWih, nWhh, nbih, nbhh, eWih, eWhh, ebih, ebhh):
    f32 = jnp.float32
    NP4 = NP // 4
    classes4 = jnp.pad(classes, ((0, NP - N), (0, 0))).reshape(NP4, 64)
    pos_p = jnp.pad(pos, ((0, NP - N), (0, 1)))
    W1a, W1b = W1[:D], W1[D:2 * D]
    W1c_p = jnp.pad(W1[2 * D:], ((0, 1), (0, 0)))
    eye4 = jnp.eye(4, dtype=f32)
    W1A4 = jnp.kron(eye4, W1a)
    W1B4 = jnp.kron(eye4, W1b)
    W2c4 = jnp.kron(eye4, W2)
    W3c4 = jnp.kron(eye4, W3)
    b1r = b1.reshape(1, D)
    b1t4 = jnp.tile(b1, 4).reshape(1, 4 * D)
    b1d4 = 2.0 * b1t4
    b2c4 = jnp.tile(b2, 4).reshape(1, 4 * D)
    b3c4 = jnp.tile(b3, 4).reshape(1, 4 * D)
    Win4 = jnp.kron(eye4, W_in)
    bin4 = jnp.tile(b_in, 4).reshape(1, 4 * D)
    Wih4 = jnp.kron(eye4, nWih)
    Whh4 = jnp.kron(eye4, nWhh)
    bih4 = jnp.tile(nbih, 4).reshape(1, 12 * D)
    bhh4 = jnp.tile(nbhh, 4).reshape(1, 12 * D)

    s_pad = jnp.pad(edges[0], (0, HP - E), constant_values=DUMMY)
    t_pad = jnp.pad(edges[1], (0, HP - E), constant_values=DUMMY)
    idx2d = jnp.concatenate([s_pad, t_pad]).reshape(IDX_ROWS, 128)
    HP2 = HP // 2
    idxA = jnp.concatenate([s_pad[:HP2], t_pad[:HP2]]).reshape(
        IDX_ROWS // 2, 128)
    idxB = jnp.concatenate([s_pad[HP2:], t_pad[HP2:]]).reshape(
        IDX_ROWS // 2, 128)
    zeros_acc = jnp.zeros((NP, D), f32)

    h0, P = pl.pallas_call(
        _setup_body,
        out_shape=(
            jax.ShapeDtypeStruct((NP4, 4 * D), f32),
            jax.ShapeDtypeStruct((NP, D), f32),
        ),
    )(classes4, pos_p, Win4, bin4, W1c_p)

    sc_gather, _ = _sc_kernels(TOT)
    _, sc_scatter_h = _sc_kernels(TOT // 2)
    GP = sc_gather(P, idx2d)
    GP4 = GP.reshape(TOT4, 4 * D)
    posterm = pl.pallas_call(
        _posf_body,
        grid=(_NB8,),
        in_specs=[
            pl.BlockSpec((_BB, 4 * D), lambda i: (i, 0)),
            pl.BlockSpec((_BB, 4 * D), lambda i: (_NB8 + i, 0)),
            pl.BlockSpec((1, 4 * D), lambda i: (0, 0)),
        ],
        out_specs=pl.BlockSpec((_BB, 4 * D), lambda i: (i, 0)),
        out_shape=jax.ShapeDtypeStruct((TOT8, 4 * D), f32),
    )(GP4, GP4, b1t4)

    aspec = [
        pl.BlockSpec((1, NP4, 4 * D), lambda i: (0, 0, 0)),
        pl.BlockSpec((1, NP4, 4 * D), lambda i: (1, 0, 0)),
        pl.BlockSpec((1, NP4, 4 * D), lambda i: (0, 0, 0)),
        pl.BlockSpec((1, NP4, 4 * D), lambda i: (1, 0, 0)),
    ]
    gru = pl.pallas_call(
        _gru_body,
        grid=(1,),
        in_specs=aspec + [
            pl.BlockSpec((NP4, 4 * D), lambda i: (0, 0)),
            pl.BlockSpec((4 * D, 12 * D), lambda i: (0, 0)),
            pl.BlockSpec((4 * D, 12 * D), lambda i: (0, 0)),
            pl.BlockSpec((1, 12 * D), lambda i: (0, 0)),
            pl.BlockSpec((1, 12 * D), lambda i: (0, 0)),
        ],
        out_specs=pl.BlockSpec((NP4, 4 * D), lambda i: (0, 0)),
        out_shape=jax.ShapeDtypeStruct((NP4, 4 * D), f32),
    )

    h4 = h0
    for _ in range(6):
        G = sc_gather(h4.reshape(NP, D), idx2d)
        G4 = G.reshape(TOT4, 4 * D)
        mA = _mlp_call(G4, posterm, W1A4, W1B4, W2c4, b2c4, W3c4, b3c4,
                       b1d4, 0)
        accA = sc_scatter_h(mA.reshape(TOT // 2, D), idxA, zeros_acc)
        mB = _mlp_call(G4, posterm, W1A4, W1B4, W2c4, b2c4, W3c4, b3c4,
                       b1d4, _NBG)
        accB = sc_scatter_h(mB.reshape(TOT // 2, D), idxB, zeros_acc)
        accA4 = accA.reshape(NC, NP4, 4 * D)
        accB4 = accB.reshape(NC, NP4, 4 * D)
        h4 = gru(accA4, accA4, accB4, accB4, h4, Wih4, Whh4, bih4, bhh4)
    return h4.reshape(NP, D)[:N]


# speedup vs baseline: 29.3659x; 1.0004x over previous
import functools

import jax
import jax.numpy as jnp
from jax import lax
from jax.experimental import pallas as pl
from jax.experimental.pallas import tpu as pltpu
from jax.experimental.pallas import tpu_sc as plsc

N = 10000
E = 320000
NP = 10016
HP = 327680
TOT = 2 * HP
D = 32
DUMMY = 10008

NC, NS = 2, 16
NW = NC * NS
PER_W = TOT // NW
IDX_ROWS = TOT // 128
ROWS_W = PER_W // 128
CHUNK_ROWS = 10
CHUNK = CHUNK_ROWS * 128
N_CHUNKS = ROWS_W // CHUNK_ROWS

@functools.lru_cache(maxsize=2)
def _sc_kernels(total):
    mesh = plsc.VectorSubcoreMesh(core_axis_name="c", subcore_axis_name="s")
    PER_W = total // NW
    ROWS_W = PER_W // 128
    N_CHUNKS = ROWS_W // CHUNK_ROWS
    M_PAIRS = N_CHUNKS // 2

    @functools.partial(
        pl.kernel,
        mesh=mesh,
        out_type=jax.ShapeDtypeStruct((total, D), jnp.float32),
        scratch_types=[
            pltpu.VMEM((ROWS_W, 128), jnp.int32),
            pltpu.VMEM((CHUNK, D), jnp.float32),
            pltpu.VMEM((CHUNK, D), jnp.float32),
            pltpu.VMEM_SHARED((NP, D), jnp.float32),
            pltpu.SemaphoreType.DMA,
            pltpu.SemaphoreType.DMA,
            pltpu.SemaphoreType.DMA,
            pltpu.SemaphoreType.DMA,
        ],
        compiler_params=pltpu.CompilerParams(use_tc_tiling_on_sc=False),
    )
    def sc_gather(table_hbm, idx_hbm, out_hbm, idx_v, rows0, rows1,
                  table_sh, semg0, semg1, semo0, semo1):
        sid = lax.axis_index("s")
        wid = sid * NC + lax.axis_index("c")
        zrows = NP // NS
        pltpu.sync_copy(
            table_hbm.at[pl.ds(sid * zrows, zrows)],
            table_sh.at[pl.ds(sid * zrows, zrows)],
        )
        pltpu.sync_copy(idx_hbm.at[pl.ds(wid * ROWS_W, ROWS_W)], idx_v)
        plsc.subcore_barrier()

        def fire(k, buf, sem):
            for j in range(CHUNK_ROWS):
                pltpu.async_copy(
                    table_sh.at[idx_v.at[k * CHUNK_ROWS + j]],
                    buf.at[pl.ds(j * 128, 128)],
                    sem,
                )

        def drain(buf, sem):
            pltpu.make_async_copy(
                table_hbm.at[pl.ds(0, CHUNK)], buf, sem).wait()

        def store(k, buf, sem):
            pltpu.async_copy(
                buf, out_hbm.at[pl.ds(wid * PER_W + k * CHUNK, CHUNK)], sem)

        def wait_store(buf, sem):
            pltpu.make_async_copy(
                buf, out_hbm.at[pl.ds(0, CHUNK)], sem).wait()

        fire(0, rows0, semg0)

        def body(m, carry):
            @pl.when(m >= 1)
            def _():
                wait_store(rows1, semo1)
            fire(2 * m + 1, rows1, semg1)
            drain(rows0, semg0)
            store(2 * m, rows0, semo0)
            @pl.when(m < M_PAIRS - 1)
            def _():
                wait_store(rows0, semo0)
                fire(2 * m + 2, rows0, semg0)
            drain(rows1, semg1)
            store(2 * m + 1, rows1, semo1)
            return carry

        lax.fori_loop(0, M_PAIRS, body, 0)
        wait_store(rows0, semo0)
        wait_store(rows1, semo1)

    @functools.partial(
        pl.kernel,
        mesh=mesh,
        out_type=jax.ShapeDtypeStruct((NC, NP, D), jnp.float32),
        scratch_types=[
            pltpu.VMEM((ROWS_W, 128), jnp.int32),
            pltpu.VMEM((CHUNK, D), jnp.float32),
            pltpu.VMEM((CHUNK, D), jnp.float32),
            pltpu.VMEM_SHARED((NP, D), jnp.float32),
            pltpu.SemaphoreType.DMA,
            pltpu.SemaphoreType.DMA,
            pltpu.SemaphoreType.DMA,
            pltpu.SemaphoreType.DMA,
        ],
        compiler_params=pltpu.CompilerParams(use_tc_tiling_on_sc=False),
    )
    def sc_scatter(msgs_hbm, idx_hbm, zeros_hbm, out_hbm, idx_v, msg0, msg1,
                   acc_sh, seml0, seml1, sems0, sems1):
        cid = lax.axis_index("c")
        sid = lax.axis_index("s")
        wid = sid * NC + cid
        zrows = NP // NS

        def load(k, buf, sem):
            pltpu.async_copy(
                msgs_hbm.at[pl.ds(wid * PER_W + k * CHUNK, CHUNK)], buf, sem)

        def wait_load(buf, sem):
            pltpu.make_async_copy(
                msgs_hbm.at[pl.ds(0, CHUNK)], buf, sem).wait()

        def fire_sc(k, buf, sem):
            for j in range(CHUNK_ROWS):
                pltpu.async_copy(
                    buf.at[pl.ds(j * 128, 128)],
                    acc_sh.at[idx_v.at[k * CHUNK_ROWS + j]],
                    sem,
                    add=True,
                )

        def drain_sc(buf, sem):
            pltpu.make_async_copy(
                msgs_hbm.at[pl.ds(0, CHUNK)], buf, sem).wait()

        load(0, msg0, seml0)
        pltpu.sync_copy(idx_hbm.at[pl.ds(wid * ROWS_W, ROWS_W)], idx_v)
        pltpu.sync_copy(
            zeros_hbm.at[pl.ds(sid * zrows, zrows)],
            acc_sh.at[pl.ds(sid * zrows, zrows)],
        )
        plsc.subcore_barrier()

        def body(m, carry):
            wait_load(msg0, seml0)
            fire_sc(2 * m, msg0, sems0)
            @pl.when(m >= 1)
            def _():
                drain_sc(msg1, sems1)
            load(2 * m + 1, msg1, seml1)
            wait_load(msg1, seml1)
            fire_sc(2 * m + 1, msg1, sems1)
            drain_sc(msg0, sems0)
            @pl.when(m < M_PAIRS - 1)
            def _():
                load(2 * m + 2, msg0, seml0)
            return carry

        lax.fori_loop(0, M_PAIRS, body, 0)
        drain_sc(msg1, sems1)
        plsc.subcore_barrier()
        pltpu.sync_copy(
            acc_sh.at[pl.ds(sid * zrows, zrows)],
            out_hbm.at[cid].at[pl.ds(sid * zrows, zrows)],
        )

    return sc_gather, sc_scatter


def _setup_body(classes_ref, pos_ref, Win_ref, bin_ref, W1c_ref, h0_ref, P_ref):
    h0_ref[...] = (
        jnp.dot(classes_ref[...], Win_ref[...],
                preferred_element_type=jnp.float32)
        + bin_ref[...]
    )
    P_ref[...] = jnp.dot(pos_ref[...], W1c_ref[...],
                         preferred_element_type=jnp.float32)


def _posf_body(gps_ref, gpt_ref, b1t_ref, out_ref):
    out_ref[...] = gpt_ref[...] - gps_ref[...] + b1t_ref[...]


def _mlp_body(gs_ref, gt_ref, posf_ref, W1A_ref, W1B_ref, W2_ref, b2_ref,
              W3_ref, b3_ref, b1d_ref, out_ref):
    hf = pl.program_id(0).astype(jnp.float32)
    posf = posf_ref[...]
    term = posf + hf * (b1d_ref[...] - 2.0 * posf)
    h1 = jnp.maximum(
        jnp.dot(gs_ref[...], W1A_ref[...], preferred_element_type=jnp.float32)
        + jnp.dot(gt_ref[...], W1B_ref[...],
                  preferred_element_type=jnp.float32)
        + term, 0.0)
    h2 = jnp.maximum(
        jnp.dot(h1, W2_ref[...], preferred_element_type=jnp.float32)
        + b2_ref[...], 0.0)
    out_ref[...] = (
        jnp.dot(h2, W3_ref[...], preferred_element_type=jnp.float32)
        + b3_ref[...]
    )


def _gru_body(aA0_ref, aA1_ref, aB0_ref, aB1_ref, h_ref, Wih_ref, Whh_ref,
              bih_ref, bhh_ref, out_ref):
    a = (aA0_ref[0] + aA1_ref[0]) + (aB0_ref[0] + aB1_ref[0])
    h = h_ref[...]
    gi = jnp.dot(a, Wih_ref[...], preferred_element_type=jnp.float32) \
        + bih_ref[...]
    gh = jnp.dot(h, Whh_ref[...], preferred_element_type=jnp.float32) \
        + bhh_ref[...]
    outs = []
    for j in range(4):
        b = 3 * D * j
        hj = h[:, D * j:D * (j + 1)]
        r = jax.nn.sigmoid(gi[:, b:b + D] + gh[:, b:b + D])
        z = jax.nn.sigmoid(gi[:, b + D:b + 2 * D] + gh[:, b + D:b + 2 * D])
        n = jnp.tanh(gi[:, b + 2 * D:b + 3 * D]
                     + r * gh[:, b + 2 * D:b + 3 * D])
        outs.append((1.0 - z) * n + z * hj)
    out_ref[...] = jnp.concatenate(outs, axis=1)


_BB = 4096
TOT4 = TOT // 4
TOT8 = TOT // 8
_NB8 = TOT8 // _BB
_NBG = _NB8 // 2


def _mlp_call(G4, posterm, W1A4, W1B4, W2c4, b2c4, W3c4, b3c4, b1d4, goff):
    wspec = lambda shape: pl.BlockSpec(shape, lambda h, i: (0, 0))
    return pl.pallas_call(
        _mlp_body,
        grid=(2, _NBG),
        in_specs=[
            pl.BlockSpec((_BB, 4 * D),
                         lambda h, i: (h * _NB8 + goff + i, 0)),
            pl.BlockSpec((_BB, 4 * D),
                         lambda h, i: ((1 - h) * _NB8 + goff + i, 0)),
            pl.BlockSpec((_BB, 4 * D), lambda h, i: (goff + i, 0)),
            wspec((4 * D, 4 * D)),
            wspec((4 * D, 4 * D)),
            wspec((4 * D, 4 * D)),
            wspec((1, 4 * D)),
            wspec((4 * D, 4 * D)),
            wspec((1, 4 * D)),
            wspec((1, 4 * D)),
        ],
        out_specs=pl.BlockSpec((_BB, 4 * D), lambda h, i: (h * _NBG + i, 0)),
        out_shape=jax.ShapeDtypeStruct((TOT8, 4 * D), jnp.float32),
    )(G4, G4, posterm, W1A4, W1B4, W2c4, b2c4, W3c4, b3c4, b1d4)


def kernel(pos, classes, edges, W_in, b_in, W1, b1, W2, b2, W3, b3,
           nWih, nWhh, nbih, nbhh, eWih, eWhh, ebih, ebhh):
    f32 = jnp.float32
    NP4 = NP // 4
    classes4 = jnp.pad(classes, ((0, NP - N), (0, 0))).reshape(NP4, 64)
    pos_p = jnp.pad(pos, ((0, NP - N), (0, 1)))
    W1a, W1b = W1[:D], W1[D:2 * D]
    W1c_p = jnp.pad(W1[2 * D:], ((0, 1), (0, 0)))
    eye4 = jnp.eye(4, dtype=f32)
    W1A4 = jnp.kron(eye4, W1a)
    W1B4 = jnp.kron(eye4, W1b)
    W2c4 = jnp.kron(eye4, W2)
    W3c4 = jnp.kron(eye4, W3)
    b1t4 = jnp.tile(b1, 4).reshape(1, 4 * D)
    b1d4 = 2.0 * b1t4
    b2c4 = jnp.tile(b2, 4).reshape(1, 4 * D)
    b3c4 = jnp.tile(b3, 4).reshape(1, 4 * D)
    Win4 = jnp.kron(eye4, W_in)
    bin4 = jnp.tile(b_in, 4).reshape(1, 4 * D)
    Wih4 = jnp.kron(eye4, nWih)
    Whh4 = jnp.kron(eye4, nWhh)
    bih4 = jnp.tile(nbih, 4).reshape(1, 12 * D)
    bhh4 = jnp.tile(nbhh, 4).reshape(1, 12 * D)

    s_pad = jnp.pad(edges[0], (0, HP - E), constant_values=DUMMY)
    t_pad = jnp.pad(edges[1], (0, HP - E), constant_values=DUMMY)
    idx2d = jnp.concatenate([s_pad, t_pad]).reshape(IDX_ROWS, 128)
    HP2 = HP // 2
    idxA = jnp.concatenate([s_pad[:HP2], t_pad[:HP2]]).reshape(
        IDX_ROWS // 2, 128)
    idxB = jnp.concatenate([s_pad[HP2:], t_pad[HP2:]]).reshape(
        IDX_ROWS // 2, 128)
    zeros_acc = jnp.zeros((NP, D), f32)

    h0, P = pl.pallas_call(
        _setup_body,
        out_shape=(
            jax.ShapeDtypeStruct((NP4, 4 * D), f32),
            jax.ShapeDtypeStruct((NP, D), f32),
        ),
    )(classes4, pos_p, Win4, bin4, W1c_p)

    sc_gather, _ = _sc_kernels(TOT)
    _, sc_scatter_h = _sc_kernels(TOT // 2)
    GP = sc_gather(P, idx2d)
    GP4 = GP.reshape(TOT4, 4 * D)
    posterm = pl.pallas_call(
        _posf_body,
        grid=(_NB8,),
        in_specs=[
            pl.BlockSpec((_BB, 4 * D), lambda i: (i, 0)),
            pl.BlockSpec((_BB, 4 * D), lambda i: (_NB8 + i, 0)),
            pl.BlockSpec((1, 4 * D), lambda i: (0, 0)),
        ],
        out_specs=pl.BlockSpec((_BB, 4 * D), lambda i: (i, 0)),
        out_shape=jax.ShapeDtypeStruct((TOT8, 4 * D), f32),
    )(GP4, GP4, b1t4)

    aspec = [
        pl.BlockSpec((1, NP4, 4 * D), lambda i: (0, 0, 0)),
        pl.BlockSpec((1, NP4, 4 * D), lambda i: (1, 0, 0)),
        pl.BlockSpec((1, NP4, 4 * D), lambda i: (0, 0, 0)),
        pl.BlockSpec((1, NP4, 4 * D), lambda i: (1, 0, 0)),
    ]
    gru = pl.pallas_call(
        _gru_body,
        grid=(1,),
        in_specs=aspec + [
            pl.BlockSpec((NP4, 4 * D), lambda i: (0, 0)),
            pl.BlockSpec((4 * D, 12 * D), lambda i: (0, 0)),
            pl.BlockSpec((4 * D, 12 * D), lambda i: (0, 0)),
            pl.BlockSpec((1, 12 * D), lambda i: (0, 0)),
            pl.BlockSpec((1, 12 * D), lambda i: (0, 0)),
        ],
        out_specs=pl.BlockSpec((NP4, 4 * D), lambda i: (0, 0)),
        out_shape=jax.ShapeDtypeStruct((NP4, 4 * D), f32),
    )

    h4 = h0
    for _ in range(6):
        G = sc_gather(h4.reshape(NP, D), idx2d)
        G4 = G.reshape(TOT4, 4 * D)
        mA = _mlp_call(G4, posterm, W1A4, W1B4, W2c4, b2c4, W3c4, b3c4,
                       b1d4, 0)
        accA = sc_scatter_h(mA.reshape(TOT // 2, D), idxA, zeros_acc)
        mB = _mlp_call(G4, posterm, W1A4, W1B4, W2c4, b2c4, W3c4, b3c4,
                       b1d4, _NBG)
        accB = sc_scatter_h(mB.reshape(TOT // 2, D), idxB, zeros_acc)
        accA4 = accA.reshape(NC, NP4, 4 * D)
        accB4 = accB.reshape(NC, NP4, 4 * D)
        h4 = gru(accA4, accA4, accB4, accB4, h4, Wih4, Whh4, bih4, bhh4)
    return h4.reshape(NP, D)[:N]


# speedup vs baseline: 29.6448x; 1.0095x over previous
import functools

import jax
import jax.numpy as jnp
from jax import lax
from jax.experimental import pallas as pl
from jax.experimental.pallas import tpu as pltpu
from jax.experimental.pallas import tpu_sc as plsc

N = 10000
E = 320000
NP = 10016
HP = 327680
TOT = 2 * HP
D = 32
DUMMY = 10008

NC, NS = 2, 16
NW = NC * NS
PER_W = TOT // NW
IDX_ROWS = TOT // 128
ROWS_W = PER_W // 128
CHUNK_ROWS = 10
CHUNK = CHUNK_ROWS * 128
N_CHUNKS = ROWS_W // CHUNK_ROWS

@functools.lru_cache(maxsize=2)
def _sc_kernels(total):
    mesh = plsc.VectorSubcoreMesh(core_axis_name="c", subcore_axis_name="s")
    PER_W = total // NW
    ROWS_W = PER_W // 128
    N_CHUNKS = ROWS_W // CHUNK_ROWS
    M_PAIRS = N_CHUNKS // 2

    @functools.partial(
        pl.kernel,
        mesh=mesh,
        out_type=jax.ShapeDtypeStruct((total, D), jnp.float32),
        scratch_types=[
            pltpu.VMEM((ROWS_W, 128), jnp.int32),
            pltpu.VMEM((CHUNK, D), jnp.float32),
            pltpu.VMEM((CHUNK, D), jnp.float32),
            pltpu.VMEM_SHARED((NP, D), jnp.float32),
            pltpu.SemaphoreType.DMA,
            pltpu.SemaphoreType.DMA,
            pltpu.SemaphoreType.DMA,
            pltpu.SemaphoreType.DMA,
        ],
        compiler_params=pltpu.CompilerParams(use_tc_tiling_on_sc=False),
    )
    def sc_gather(table_hbm, idx_hbm, out_hbm, idx_v, rows0, rows1,
                  table_sh, semg0, semg1, semo0, semo1):
        sid = lax.axis_index("s")
        wid = sid * NC + lax.axis_index("c")
        zrows = NP // NS
        pltpu.sync_copy(
            table_hbm.at[pl.ds(sid * zrows, zrows)],
            table_sh.at[pl.ds(sid * zrows, zrows)],
        )
        pltpu.sync_copy(idx_hbm.at[pl.ds(wid * ROWS_W, ROWS_W)], idx_v)
        plsc.subcore_barrier()

        def fire(k, buf, sem):
            for j in range(CHUNK_ROWS):
                pltpu.async_copy(
                    table_sh.at[idx_v.at[k * CHUNK_ROWS + j]],
                    buf.at[pl.ds(j * 128, 128)],
                    sem,
                )

        def drain(buf, sem):
            pltpu.make_async_copy(
                table_hbm.at[pl.ds(0, CHUNK)], buf, sem).wait()

        def store(k, buf, sem):
            pltpu.async_copy(
                buf, out_hbm.at[pl.ds(wid * PER_W + k * CHUNK, CHUNK)], sem)

        def wait_store(buf, sem):
            pltpu.make_async_copy(
                buf, out_hbm.at[pl.ds(0, CHUNK)], sem).wait()

        fire(0, rows0, semg0)

        def body(m, carry):
            @pl.when(m >= 1)
            def _():
                wait_store(rows1, semo1)
            fire(2 * m + 1, rows1, semg1)
            drain(rows0, semg0)
            store(2 * m, rows0, semo0)
            @pl.when(m < M_PAIRS - 1)
            def _():
                wait_store(rows0, semo0)
                fire(2 * m + 2, rows0, semg0)
            drain(rows1, semg1)
            store(2 * m + 1, rows1, semo1)
            return carry

        lax.fori_loop(0, M_PAIRS, body, 0)
        wait_store(rows0, semo0)
        wait_store(rows1, semo1)

    @functools.partial(
        pl.kernel,
        mesh=mesh,
        out_type=jax.ShapeDtypeStruct((NC, NP, D), jnp.float32),
        scratch_types=[
            pltpu.VMEM((ROWS_W, 128), jnp.int32),
            pltpu.VMEM((CHUNK, D), jnp.float32),
            pltpu.VMEM((CHUNK, D), jnp.float32),
            pltpu.VMEM_SHARED((NP, D), jnp.float32),
            pltpu.SemaphoreType.DMA,
            pltpu.SemaphoreType.DMA,
            pltpu.SemaphoreType.DMA,
            pltpu.SemaphoreType.DMA,
        ],
        compiler_params=pltpu.CompilerParams(use_tc_tiling_on_sc=False),
    )
    def sc_scatter(msgs_hbm, idx_hbm, zeros_hbm, out_hbm, idx_v, msg0, msg1,
                   acc_sh, seml0, seml1, sems0, sems1):
        cid = lax.axis_index("c")
        sid = lax.axis_index("s")
        wid = sid * NC + cid
        zrows = NP // NS

        def load(k, buf, sem):
            pltpu.async_copy(
                msgs_hbm.at[pl.ds(wid * PER_W + k * CHUNK, CHUNK)], buf, sem)

        def wait_load(buf, sem):
            pltpu.make_async_copy(
                msgs_hbm.at[pl.ds(0, CHUNK)], buf, sem).wait()

        def fire_sc(k, buf, sem):
            for j in range(CHUNK_ROWS):
                pltpu.async_copy(
                    buf.at[pl.ds(j * 128, 128)],
                    acc_sh.at[idx_v.at[k * CHUNK_ROWS + j]],
                    sem,
                    add=True,
                )

        def drain_sc(buf, sem):
            pltpu.make_async_copy(
                msgs_hbm.at[pl.ds(0, CHUNK)], buf, sem).wait()

        load(0, msg0, seml0)
        pltpu.sync_copy(idx_hbm.at[pl.ds(wid * ROWS_W, ROWS_W)], idx_v)
        pltpu.sync_copy(
            zeros_hbm.at[pl.ds(sid * zrows, zrows)],
            acc_sh.at[pl.ds(sid * zrows, zrows)],
        )
        plsc.subcore_barrier()

        def body(m, carry):
            wait_load(msg0, seml0)
            fire_sc(2 * m, msg0, sems0)
            @pl.when(m >= 1)
            def _():
                drain_sc(msg1, sems1)
            load(2 * m + 1, msg1, seml1)
            wait_load(msg1, seml1)
            fire_sc(2 * m + 1, msg1, sems1)
            drain_sc(msg0, sems0)
            @pl.when(m < M_PAIRS - 1)
            def _():
                load(2 * m + 2, msg0, seml0)
            return carry

        lax.fori_loop(0, M_PAIRS, body, 0)
        drain_sc(msg1, sems1)
        plsc.subcore_barrier()
        pltpu.sync_copy(
            acc_sh.at[pl.ds(sid * zrows, zrows)],
            out_hbm.at[cid].at[pl.ds(sid * zrows, zrows)],
        )

    return sc_gather, sc_scatter


def _setup_body(classes_ref, pos_ref, Win_ref, bin_ref, W1c_ref, h0_ref, P_ref):
    h0_ref[...] = (
        jnp.dot(classes_ref[...], Win_ref[...],
                preferred_element_type=jnp.float32)
        + bin_ref[...]
    )
    P_ref[...] = jnp.dot(pos_ref[...], W1c_ref[...],
                         preferred_element_type=jnp.float32)


def _posf_body(gps_ref, gpt_ref, b1t_ref, out_ref):
    out_ref[...] = gpt_ref[...] - gps_ref[...] + b1t_ref[...]


def _mlp_body(gs_ref, gt_ref, posf_ref, W1A_ref, W1B_ref, W2_ref, b2_ref,
              W3_ref, b3_ref, b1d_ref, out_ref):
    hf = pl.program_id(0).astype(jnp.float32)
    posf = posf_ref[...]
    term = posf + hf * (b1d_ref[...] - 2.0 * posf)
    h1 = jnp.maximum(
        jnp.dot(gs_ref[...], W1A_ref[...], preferred_element_type=jnp.float32)
        + jnp.dot(gt_ref[...], W1B_ref[...],
                  preferred_element_type=jnp.float32)
        + term, 0.0)
    h2 = jnp.maximum(
        jnp.dot(h1, W2_ref[...], preferred_element_type=jnp.float32)
        + b2_ref[...], 0.0)
    out_ref[...] = (
        jnp.dot(h2, W3_ref[...], preferred_element_type=jnp.float32)
        + b3_ref[...]
    )


def _gru_body(aA0_ref, aA1_ref, aB0_ref, aB1_ref, h_ref, Wih_ref, Whh_ref,
              bih_ref, bhh_ref, out_ref):
    a = (aA0_ref[0] + aA1_ref[0]) + (aB0_ref[0] + aB1_ref[0])
    h = h_ref[...]
    gi = jnp.dot(a, Wih_ref[...], preferred_element_type=jnp.float32) \
        + bih_ref[...]
    gh = jnp.dot(h, Whh_ref[...], preferred_element_type=jnp.float32) \
        + bhh_ref[...]
    outs = []
    for j in range(4):
        b = 3 * D * j
        hj = h[:, D * j:D * (j + 1)]
        r = jax.nn.sigmoid(gi[:, b:b + D] + gh[:, b:b + D])
        z = jax.nn.sigmoid(gi[:, b + D:b + 2 * D] + gh[:, b + D:b + 2 * D])
        n = jnp.tanh(gi[:, b + 2 * D:b + 3 * D]
                     + r * gh[:, b + 2 * D:b + 3 * D])
        outs.append((1.0 - z) * n + z * hj)
    out_ref[...] = jnp.concatenate(outs, axis=1)


_BB = 8192
TOT4 = TOT // 4
TOT8 = TOT // 8
_NB8 = TOT8 // _BB
_NBG = _NB8 // 2


def _mlp_call(G4, posterm, W1A4, W1B4, W2c4, b2c4, W3c4, b3c4, b1d4, goff):
    wspec = lambda shape: pl.BlockSpec(shape, lambda h, i: (0, 0))
    return pl.pallas_call(
        _mlp_body,
        grid=(2, _NBG),
        in_specs=[
            pl.BlockSpec((_BB, 4 * D),
                         lambda h, i: (h * _NB8 + goff + i, 0)),
            pl.BlockSpec((_BB, 4 * D),
                         lambda h, i: ((1 - h) * _NB8 + goff + i, 0)),
            pl.BlockSpec((_BB, 4 * D), lambda h, i: (goff + i, 0)),
            wspec((4 * D, 4 * D)),
            wspec((4 * D, 4 * D)),
            wspec((4 * D, 4 * D)),
            wspec((1, 4 * D)),
            wspec((4 * D, 4 * D)),
            wspec((1, 4 * D)),
            wspec((1, 4 * D)),
        ],
        out_specs=pl.BlockSpec((_BB, 4 * D), lambda h, i: (h * _NBG + i, 0)),
        out_shape=jax.ShapeDtypeStruct((TOT8, 4 * D), jnp.float32),
    )(G4, G4, posterm, W1A4, W1B4, W2c4, b2c4, W3c4, b3c4, b1d4)


def kernel(pos, classes, edges, W_in, b_in, W1, b1, W2, b2, W3, b3,
           nWih, nWhh, nbih, nbhh, eWih, eWhh, ebih, ebhh):
    f32 = jnp.float32
    NP4 = NP // 4
    classes4 = jnp.pad(classes, ((0, NP - N), (0, 0))).reshape(NP4, 64)
    pos_p = jnp.pad(pos, ((0, NP - N), (0, 1)))
    W1a, W1b = W1[:D], W1[D:2 * D]
    W1c_p = jnp.pad(W1[2 * D:], ((0, 1), (0, 0)))
    eye4 = jnp.eye(4, dtype=f32)
    W1A4 = jnp.kron(eye4, W1a)
    W1B4 = jnp.kron(eye4, W1b)
    W2c4 = jnp.kron(eye4, W2)
    W3c4 = jnp.kron(eye4, W3)
    b1t4 = jnp.tile(b1, 4).reshape(1, 4 * D)
    b1d4 = 2.0 * b1t4
    b2c4 = jnp.tile(b2, 4).reshape(1, 4 * D)
    b3c4 = jnp.tile(b3, 4).reshape(1, 4 * D)
    Win4 = jnp.kron(eye4, W_in)
    bin4 = jnp.tile(b_in, 4).reshape(1, 4 * D)
    Wih4 = jnp.kron(eye4, nWih)
    Whh4 = jnp.kron(eye4, nWhh)
    bih4 = jnp.tile(nbih, 4).reshape(1, 12 * D)
    bhh4 = jnp.tile(nbhh, 4).reshape(1, 12 * D)

    s_pad = jnp.pad(edges[0], (0, HP - E), constant_values=DUMMY)
    t_pad = jnp.pad(edges[1], (0, HP - E), constant_values=DUMMY)
    idx2d = jnp.concatenate([s_pad, t_pad]).reshape(IDX_ROWS, 128)
    HP2 = HP // 2
    idxA = jnp.concatenate([s_pad[:HP2], t_pad[:HP2]]).reshape(
        IDX_ROWS // 2, 128)
    idxB = jnp.concatenate([s_pad[HP2:], t_pad[HP2:]]).reshape(
        IDX_ROWS // 2, 128)
    zeros_acc = jnp.zeros((NP, D), f32)

    h0, P = pl.pallas_call(
        _setup_body,
        out_shape=(
            jax.ShapeDtypeStruct((NP4, 4 * D), f32),
            jax.ShapeDtypeStruct((NP, D), f32),
        ),
    )(classes4, pos_p, Win4, bin4, W1c_p)

    sc_gather, _ = _sc_kernels(TOT)
    _, sc_scatter_h = _sc_kernels(TOT // 2)
    GP = sc_gather(P, idx2d)
    GP4 = GP.reshape(TOT4, 4 * D)
    posterm = pl.pallas_call(
        _posf_body,
        grid=(_NB8,),
        in_specs=[
            pl.BlockSpec((_BB, 4 * D), lambda i: (i, 0)),
            pl.BlockSpec((_BB, 4 * D), lambda i: (_NB8 + i, 0)),
            pl.BlockSpec((1, 4 * D), lambda i: (0, 0)),
        ],
        out_specs=pl.BlockSpec((_BB, 4 * D), lambda i: (i, 0)),
        out_shape=jax.ShapeDtypeStruct((TOT8, 4 * D), f32),
    )(GP4, GP4, b1t4)

    aspec = [
        pl.BlockSpec((1, NP4, 4 * D), lambda i: (0, 0, 0)),
        pl.BlockSpec((1, NP4, 4 * D), lambda i: (1, 0, 0)),
        pl.BlockSpec((1, NP4, 4 * D), lambda i: (0, 0, 0)),
        pl.BlockSpec((1, NP4, 4 * D), lambda i: (1, 0, 0)),
    ]
    gru = pl.pallas_call(
        _gru_body,
        grid=(1,),
        in_specs=aspec + [
            pl.BlockSpec((NP4, 4 * D), lambda i: (0, 0)),
            pl.BlockSpec((4 * D, 12 * D), lambda i: (0, 0)),
            pl.BlockSpec((4 * D, 12 * D), lambda i: (0, 0)),
            pl.BlockSpec((1, 12 * D), lambda i: (0, 0)),
            pl.BlockSpec((1, 12 * D), lambda i: (0, 0)),
        ],
        out_specs=pl.BlockSpec((NP4, 4 * D), lambda i: (0, 0)),
        out_shape=jax.ShapeDtypeStruct((NP4, 4 * D), f32),
    )

    h4 = h0
    for _ in range(6):
        G = sc_gather(h4.reshape(NP, D), idx2d)
        G4 = G.reshape(TOT4, 4 * D)
        mA = _mlp_call(G4, posterm, W1A4, W1B4, W2c4, b2c4, W3c4, b3c4,
                       b1d4, 0)
        accA = sc_scatter_h(mA.reshape(TOT // 2, D), idxA, zeros_acc)
        mB = _mlp_call(G4, posterm, W1A4, W1B4, W2c4, b2c4, W3c4, b3c4,
                       b1d4, _NBG)
        accB = sc_scatter_h(mB.reshape(TOT // 2, D), idxB, zeros_acc)
        accA4 = accA.reshape(NC, NP4, 4 * D)
        accB4 = accB.reshape(NC, NP4, 4 * D)
        h4 = gru(accA4, accA4, accB4, accB4, h4, Wih4, Whh4, bih4, bhh4)
    return h4.reshape(NP, D)[:N]
